# Initial kernel scaffold; baseline (speedup 1.0000x reference)
import functools

import jax
import jax.numpy as jnp
from jax import lax
from jax.experimental import pallas as pl
from jax.experimental.pallas import tpu as pltpu
from jax.experimental.pallas import tpu_sc as plsc

N = 10000
E = 320000
NFEAT = 128
H = 12
C = 16
HC = H * C

BN = 400
GN = N // BN
BE = 3200
GE = E // BE

NSC = 2
NTILE = 16
NW = NSC * NTILE
EPT = E // NW
CH = 80
NCH = EPT // CH
RPT = N // NTILE
WB = 125
NWB = RPT // WB

_f32 = jnp.float32


def _blockdiag_ones(rows, cols):
    r = lax.broadcasted_iota(jnp.int32, (rows, cols), 0)
    c = lax.broadcasted_iota(jnp.int32, (rows, cols), 1)
    return (r // C == c).astype(_f32)


def _blockdiag_ones_t(rows, cols):
    r = lax.broadcasted_iota(jnp.int32, (rows, cols), 0)
    c = lax.broadcasted_iota(jnp.int32, (rows, cols), 1)
    return (c // C == r).astype(_f32)


def _node_prologue_body(x_ref, w_ref, atts_ref, attd_ref,
                        xw_ref, as_ref, ad_ref, ms_ref, md_ref):
    i = pl.program_id(0)
    xb = x_ref[...]
    xw = jnp.dot(xb, w_ref[...], preferred_element_type=_f32)
    xw_ref[...] = xw
    B = _blockdiag_ones(HC, C)
    a_s = jnp.dot(xw * atts_ref[...], B, preferred_element_type=_f32)
    a_d = jnp.dot(xw * attd_ref[...], B, preferred_element_type=_f32)
    as_ref[...] = a_s
    ad_ref[...] = a_d
    bs = jnp.broadcast_to(jnp.max(a_s, axis=0, keepdims=True), (8, C))
    bd = jnp.broadcast_to(jnp.max(a_d, axis=0, keepdims=True), (8, C))

    @pl.when(i == 0)
    def _():
        ms_ref[...] = bs
        md_ref[...] = bd

    @pl.when(i > 0)
    def _():
        ms_ref[...] = jnp.maximum(ms_ref[...], bs)
        md_ref[...] = jnp.maximum(md_ref[...], bd)


def _tc_node_prologue(x, w, attsf, attdf):
    return pl.pallas_call(
        _node_prologue_body,
        grid=(GN,),
        in_specs=[
            pl.BlockSpec((BN, NFEAT), lambda i: (i, 0)),
            pl.BlockSpec((NFEAT, HC), lambda i: (0, 0)),
            pl.BlockSpec((1, HC), lambda i: (0, 0)),
            pl.BlockSpec((1, HC), lambda i: (0, 0)),
        ],
        out_specs=[
            pl.BlockSpec((BN, HC), lambda i: (i, 0)),
            pl.BlockSpec((BN, C), lambda i: (i, 0)),
            pl.BlockSpec((BN, C), lambda i: (i, 0)),
            pl.BlockSpec((8, C), lambda i: (0, 0)),
            pl.BlockSpec((8, C), lambda i: (0, 0)),
        ],
        out_shape=[
            jax.ShapeDtypeStruct((N, HC), _f32),
            jax.ShapeDtypeStruct((N, C), _f32),
            jax.ShapeDtypeStruct((N, C), _f32),
            jax.ShapeDtypeStruct((8, C), _f32),
            jax.ShapeDtypeStruct((8, C), _f32),
        ],
    )(x, w, attsf, attdf)


def _edge_prologue_body(ea_ref, we_ref, atte_ref, ae_ref, me_ref):
    i = pl.program_id(0)
    ew = jnp.dot(ea_ref[...], we_ref[...], preferred_element_type=_f32)
    B = _blockdiag_ones(HC, C)
    ae = jnp.dot(ew * atte_ref[...], B, preferred_element_type=_f32)
    col = lax.broadcasted_iota(jnp.int32, (BE, C), 1)
    ae = ae + jnp.where(col == H, 1.0, 0.0).astype(_f32)
    ae_ref[...] = ae
    bm = jnp.broadcast_to(jnp.max(ae, axis=0, keepdims=True), (8, C))

    @pl.when(i == 0)
    def _():
        me_ref[...] = bm

    @pl.when(i > 0)
    def _():
        me_ref[...] = jnp.maximum(me_ref[...], bm)


def _tc_edge_prologue(ea, we, attef):
    return pl.pallas_call(
        _edge_prologue_body,
        grid=(GE,),
        in_specs=[
            pl.BlockSpec((BE, C), lambda i: (i, 0)),
            pl.BlockSpec((C, HC), lambda i: (0, 0)),
            pl.BlockSpec((1, HC), lambda i: (0, 0)),
        ],
        out_specs=[
            pl.BlockSpec((BE, C), lambda i: (i, 0)),
            pl.BlockSpec((8, C), lambda i: (0, 0)),
        ],
        out_shape=[
            jax.ShapeDtypeStruct((E, C), _f32),
            jax.ShapeDtypeStruct((8, C), _f32),
        ],
    )(ea, we, attef)


def _sc_loopsum_body(rows_hbm, idx_hbm, out_hbm, idxb, rowsb, wbuf, acc):
    c = lax.axis_index("c")
    s = lax.axis_index("s")
    tid = s * NSC + c

    def zrow(i, _):
        wbuf[i, :] = jnp.zeros((C,), _f32)
        return 0

    lax.fori_loop(0, WB, zrow, 0)
    for k in range(NWB):
        pltpu.sync_copy(wbuf, acc.at[pl.ds(s * RPT + k * WB, WB)])
    plsc.subcore_barrier()

    def chunk(j, _):
        base = tid * EPT + j * CH
        pltpu.sync_copy(idx_hbm.at[pl.ds(base, CH)], idxb)
        pltpu.sync_copy(rows_hbm.at[pl.ds(base, CH)], rowsb)
        pltpu.sync_copy(rowsb, acc.at[idxb], add=True)
        return 0

    lax.fori_loop(0, NCH, chunk, 0)
    plsc.subcore_barrier()
    for k in range(NWB):
        sl = pl.ds(s * RPT + k * WB, WB)
        pltpu.sync_copy(acc.at[sl], wbuf)
        pltpu.sync_copy(wbuf, out_hbm.at[c, sl])


_sc_loopsum = pl.kernel(
    _sc_loopsum_body,
    out_type=jax.ShapeDtypeStruct((NSC, N, C), _f32),
    mesh=plsc.VectorSubcoreMesh(core_axis_name="c", subcore_axis_name="s"),
    scratch_types=[
        pltpu.VMEM((CH,), jnp.int32),
        pltpu.VMEM((CH, C), _f32),
        pltpu.VMEM((WB, C), _f32),
        pltpu.VMEM_SHARED((N, C), _f32),
    ],
)


def _sc_edge_pass_body(has_edge, *refs):
    if has_edge:
        (src_hbm, dst_hbm, as_hbm, ad_hbm, ae_hbm, xw_hbm, ms_hbm, md_hbm,
         me_hbm, num_out, den_out, idxs, idxd, rs, rd, re, xwr, outr, exr,
         m1, m2, m3, wbuf, wbden, accn, accd, sem1, sem2, sem3) = refs
    else:
        (src_hbm, dst_hbm, as_hbm, ad_hbm, xw_hbm, ms_hbm, md_hbm,
         num_out, den_out, idxs, idxd, rs, rd, xwr, outr, exr,
         m1, m2, m3, wbuf, wbden, accn, accd, sem1, sem2, sem3) = refs
        re = None
    c = lax.axis_index("c")
    s = lax.axis_index("s")
    tid = s * NSC + c

    pltpu.sync_copy(ms_hbm, m1)
    pltpu.sync_copy(md_hbm, m2)
    S = m1[0, :] + m2[0, :]
    if has_edge:
        pltpu.sync_copy(me_hbm, m3)
        S = S + m3[0, :]
    S = jnp.maximum(S, 0.0)

    def zrow(i, _):
        for j in range(H):
            wbuf[i, pl.ds(j * C, C)] = jnp.zeros((C,), _f32)
        wbden[i, :] = jnp.zeros((C,), _f32)
        return 0

    lax.fori_loop(0, WB, zrow, 0)
    for k in range(NWB):
        sl = pl.ds(s * RPT + k * WB, WB)
        pltpu.sync_copy(wbuf, accn.at[sl])
        pltpu.sync_copy(wbden, accd.at[sl])
    plsc.subcore_barrier()

    def chunk(j, _):
        base = tid * EPT + j * CH
        pltpu.sync_copy(src_hbm.at[pl.ds(base, CH)], idxs)
        pltpu.sync_copy(dst_hbm.at[pl.ds(base, CH)], idxd)
        cp1 = pltpu.async_copy(as_hbm.at[idxs], rs, sem1)
        cp2 = pltpu.async_copy(ad_hbm.at[idxd], rd, sem2)
        cp3 = pltpu.async_copy(xw_hbm.at[idxs], xwr, sem3)
        if has_edge:
            pltpu.sync_copy(ae_hbm.at[pl.ds(base, CH)], re)
        cp1.wait()
        cp2.wait()
        cp3.wait()

        def edge(e, _):
            a = rs[e, :] + rd[e, :]
            if has_edge:
                a = a + re[e, :]
            a = jnp.where(a >= 0.0, a, 0.2 * a)
            ex = jnp.exp(a - S)
            exr[e, :] = ex
            for h in range(H):
                b = jnp.full((C,), exr[e, h], _f32)
                outr[e, pl.ds(h * C, C)] = xwr[e, pl.ds(h * C, C)] * b
            return 0

        lax.fori_loop(0, CH, edge, 0)
        pltpu.sync_copy(outr, accn.at[idxd], add=True)
        pltpu.sync_copy(exr, accd.at[idxd], add=True)
        return 0

    lax.fori_loop(0, NCH, chunk, 0)
    plsc.subcore_barrier()
    for k in range(NWB):
        sl = pl.ds(s * RPT + k * WB, WB)
        pltpu.sync_copy(accn.at[sl], wbuf)
        pltpu.sync_copy(wbuf, num_out.at[c, sl])
        pltpu.sync_copy(accd.at[sl], wbden)
        pltpu.sync_copy(wbden, den_out.at[c, sl])


def _make_sc_edge_pass(has_edge):
    scratch = [
        pltpu.VMEM((CH,), jnp.int32),
        pltpu.VMEM((CH,), jnp.int32),
        pltpu.VMEM((CH, C), _f32),
        pltpu.VMEM((CH, C), _f32),
    ]
    if has_edge:
        scratch.append(pltpu.VMEM((CH, C), _f32))
    scratch += [
        pltpu.VMEM((CH, HC), _f32),
        pltpu.VMEM((CH, HC), _f32),
        pltpu.VMEM((CH, C), _f32),
        pltpu.VMEM((8, C), _f32),
        pltpu.VMEM((8, C), _f32),
        pltpu.VMEM((8, C), _f32),
        pltpu.VMEM((WB, HC), _f32),
        pltpu.VMEM((WB, C), _f32),
        pltpu.VMEM_SHARED((N, HC), _f32),
        pltpu.VMEM_SHARED((N, C), _f32),
        pltpu.SemaphoreType.DMA,
        pltpu.SemaphoreType.DMA,
        pltpu.SemaphoreType.DMA,
    ]
    return pl.kernel(
        functools.partial(_sc_edge_pass_body, has_edge),
        out_type=(
            jax.ShapeDtypeStruct((NSC, N, HC), _f32),
            jax.ShapeDtypeStruct((NSC, N, C), _f32),
        ),
        mesh=plsc.VectorSubcoreMesh(core_axis_name="c", subcore_axis_name="s"),
        scratch_types=scratch,
    )


_sc_edge_pass1 = _make_sc_edge_pass(True)
_sc_edge_pass2 = _make_sc_edge_pass(False)


def _combine1_body(a0_ref, a1_ref, d0_ref, d1_ref, l0_ref, l1_ref,
                   as_ref, ad_ref, xw_ref, ms_ref, md_ref, me_ref,
                   b1_ref, w2_ref, atts2_ref, attd2_ref,
                   xw2_ref, as2_ref, ad2_ref, ms2_ref, md2_ref):
    i = pl.program_id(0)
    num = a0_ref[...] + a1_ref[...]
    den = d0_ref[...] + d1_ref[...]
    ls = l0_ref[...] + l1_ref[...]
    onehot12 = jnp.where(
        lax.broadcasted_iota(jnp.int32, (1, C), 1) == H, 1.0, 0.0).astype(_f32)
    cnt = jnp.sum(ls * onehot12, axis=1, keepdims=True)
    lae = ls / jnp.maximum(cnt, 1.0)
    S = jnp.maximum(ms_ref[0:1, :] + md_ref[0:1, :] + me_ref[0:1, :], 0.0)
    al = as_ref[...] + ad_ref[...] + lae
    al = jnp.where(al >= 0.0, al, 0.2 * al)
    exl = jnp.exp(al - S)
    R = _blockdiag_ones_t(C, HC)
    exb = jnp.dot(exl, R, preferred_element_type=_f32)
    den_t = den + exl
    num_t = num + xw_ref[...] * exb
    dinv = 1.0 / (den_t + 1e-16)
    dinvb = jnp.dot(dinv, R, preferred_element_type=_f32)
    h = num_t * dinvb + b1_ref[...]
    h = jnp.where(h > 0.0, h, jnp.exp(h) - 1.0)
    xw2 = jnp.dot(h, w2_ref[...], preferred_element_type=_f32)
    xw2_ref[...] = xw2
    B = _blockdiag_ones(HC, C)
    a_s2 = jnp.dot(xw2 * atts2_ref[...], B, preferred_element_type=_f32)
    a_d2 = jnp.dot(xw2 * attd2_ref[...], B, preferred_element_type=_f32)
    as2_ref[...] = a_s2
    ad2_ref[...] = a_d2
    bs = jnp.broadcast_to(jnp.max(a_s2, axis=0, keepdims=True), (8, C))
    bd = jnp.broadcast_to(jnp.max(a_d2, axis=0, keepdims=True), (8, C))

    @pl.when(i == 0)
    def _():
        ms2_ref[...] = bs
        md2_ref[...] = bd

    @pl.when(i > 0)
    def _():
        ms2_ref[...] = jnp.maximum(ms2_ref[...], bs)
        md2_ref[...] = jnp.maximum(md2_ref[...], bd)


def _tc_combine1(a0, a1, d0, d1, l0, l1, asrc, adst, xw, ms, md, me,
                 b1r, w2, atts2f, attd2f):
    bspec_n192 = pl.BlockSpec((BN, HC), lambda i: (i, 0))
    bspec_n16 = pl.BlockSpec((BN, C), lambda i: (i, 0))
    bspec_m = pl.BlockSpec((8, C), lambda i: (0, 0))
    return pl.pallas_call(
        _combine1_body,
        grid=(GN,),
        in_specs=[
            bspec_n192, bspec_n192, bspec_n16, bspec_n16,
            bspec_n16, bspec_n16, bspec_n16, bspec_n16, bspec_n192,
            bspec_m, bspec_m, bspec_m,
            pl.BlockSpec((1, HC), lambda i: (0, 0)),
            pl.BlockSpec((HC, HC), lambda i: (0, 0)),
            pl.BlockSpec((1, HC), lambda i: (0, 0)),
            pl.BlockSpec((1, HC), lambda i: (0, 0)),
        ],
        out_specs=[bspec_n192, bspec_n16, bspec_n16, bspec_m, bspec_m],
        out_shape=[
            jax.ShapeDtypeStruct((N, HC), _f32),
            jax.ShapeDtypeStruct((N, C), _f32),
            jax.ShapeDtypeStruct((N, C), _f32),
            jax.ShapeDtypeStruct((8, C), _f32),
            jax.ShapeDtypeStruct((8, C), _f32),
        ],
    )(a0, a1, d0, d1, l0, l1, asrc, adst, xw, ms, md, me,
      b1r, w2, atts2f, attd2f)


def _combine2_body(a0_ref, a1_ref, d0_ref, d1_ref, as_ref, ad_ref, xw_ref,
                   ms_ref, md_ref, b2_ref, h2_ref, lp_ref):
    num = a0_ref[...] + a1_ref[...]
    den = d0_ref[...] + d1_ref[...]
    S = jnp.maximum(ms_ref[0:1, :] + md_ref[0:1, :], 0.0)
    al = as_ref[...] + ad_ref[...]
    al = jnp.where(al >= 0.0, al, 0.2 * al)
    exl = jnp.exp(al - S)
    R = _blockdiag_ones_t(C, HC)
    exb = jnp.dot(exl, R, preferred_element_type=_f32)
    den_t = den + exl
    num_t = num + xw_ref[...] * exb
    dinv = 1.0 / (den_t + 1e-16)
    dinvb = jnp.dot(dinv, R, preferred_element_type=_f32)
    out = num_t * dinvb
    rr = lax.broadcasted_iota(jnp.int32, (HC, C), 0)
    cc = lax.broadcasted_iota(jnp.int32, (HC, C), 1)
    Rm = jnp.where(rr % C == cc, 1.0 / H, 0.0).astype(_f32)
    h2 = jnp.dot(out, Rm, preferred_element_type=_f32) + b2_ref[...]
    h2_ref[...] = h2
    m = jnp.max(h2, axis=1, keepdims=True)
    z = h2 - m
    lse = jnp.log(jnp.sum(jnp.exp(z), axis=1, keepdims=True))
    lp_ref[...] = z - lse


def _tc_combine2(a0, a1, d0, d1, asrc, adst, xw, ms, md, b2r):
    bspec_n192 = pl.BlockSpec((BN, HC), lambda i: (i, 0))
    bspec_n16 = pl.BlockSpec((BN, C), lambda i: (i, 0))
    bspec_m = pl.BlockSpec((8, C), lambda i: (0, 0))
    return pl.pallas_call(
        _combine2_body,
        grid=(GN,),
        in_specs=[
            bspec_n192, bspec_n192, bspec_n16, bspec_n16,
            bspec_n16, bspec_n16, bspec_n192,
            bspec_m, bspec_m,
            pl.BlockSpec((1, C), lambda i: (0, 0)),
        ],
        out_specs=[bspec_n16, bspec_n16],
        out_shape=[
            jax.ShapeDtypeStruct((N, C), _f32),
            jax.ShapeDtypeStruct((N, C), _f32),
        ],
    )(a0, a1, d0, d1, asrc, adst, xw, ms, md, b2r)


def kernel(x, edge_index, edge_attr, W1, att_src1, att_dst1, W_edge1,
           att_edge1, bias1, W2, att_src2, att_dst2, bias2):
    src = edge_index[0]
    dst = edge_index[1]
    attsf1 = att_src1.reshape(1, HC)
    attdf1 = att_dst1.reshape(1, HC)
    attef1 = att_edge1.reshape(1, HC)
    attsf2 = att_src2.reshape(1, HC)
    attdf2 = att_dst2.reshape(1, HC)

    xw1, asrc1, adst1, ms1, md1 = _tc_node_prologue(x, W1, attsf1, attdf1)
    ae1, me1 = _tc_edge_prologue(edge_attr, W_edge1, attef1)
    loops = _sc_loopsum(ae1, dst)
    nump1, denp1 = _sc_edge_pass1(src, dst, asrc1, adst1, ae1, xw1,
                                  ms1, md1, me1)
    xw2, asrc2, adst2, ms2, md2 = _tc_combine1(
        nump1[0], nump1[1], denp1[0], denp1[1], loops[0], loops[1],
        asrc1, adst1, xw1, ms1, md1, me1,
        bias1.reshape(1, HC), W2, attsf2, attdf2)
    nump2, denp2 = _sc_edge_pass2(src, dst, asrc2, adst2, xw2, ms2, md2)
    h2, lp = _tc_combine2(nump2[0], nump2[1], denp2[0], denp2[1],
                          asrc2, adst2, xw2, ms2, md2, bias2.reshape(1, C))
    return (h2, lp)


# speedup vs baseline: 25.5498x; 25.5498x over previous
import functools

import jax
import jax.numpy as jnp
from jax import lax
from jax.experimental import pallas as pl
from jax.experimental.pallas import tpu as pltpu
from jax.experimental.pallas import tpu_sc as plsc

N = 10000
E = 320000
NFEAT = 128
H = 12
C = 16
HC = H * C
HH = H // 2
HW = HH * C

BN = 400
GN = N // BN
BE = 3200
GE = E // BE

NSC = 2
NTILE = 16
NW = NSC * NTILE
CH = 80
EPT_A = E // NW
NCH_A = EPT_A // CH
EPT_B = E // NTILE
NCH_B = EPT_B // CH
RPT = N // NTILE
WB = 125
NWB = RPT // WB

_f32 = jnp.float32

_SC_PARAMS = pltpu.CompilerParams(
    use_tc_tiling_on_sc=False, needs_layout_passes=False)


def _bd(rows, cols, shift):
    r = lax.broadcasted_iota(jnp.int32, (rows, cols), 0)
    c = lax.broadcasted_iota(jnp.int32, (rows, cols), 1)
    return (r // C + shift == c).astype(_f32)


def _bd_t(rows, cols, shift):
    r = lax.broadcasted_iota(jnp.int32, (rows, cols), 0)
    c = lax.broadcasted_iota(jnp.int32, (rows, cols), 1)
    return (c // C + shift == r).astype(_f32)


def _node_prologue_body(x_ref, wlo_ref, whi_ref, aslo_ref, ashi_ref,
                        adlo_ref, adhi_ref,
                        xw_ref, as_ref, ad_ref, ms_ref, md_ref):
    i = pl.program_id(0)
    xb = x_ref[...]
    xwlo = jnp.dot(xb, wlo_ref[...], preferred_element_type=_f32)
    xwhi = jnp.dot(xb, whi_ref[...], preferred_element_type=_f32)
    xw_ref[0] = xwlo
    xw_ref[1] = xwhi
    Blo = _bd(HW, C, 0)
    Bhi = _bd(HW, C, HH)
    a_s = (jnp.dot(xwlo * aslo_ref[...], Blo, preferred_element_type=_f32)
           + jnp.dot(xwhi * ashi_ref[...], Bhi, preferred_element_type=_f32))
    a_d = (jnp.dot(xwlo * adlo_ref[...], Blo, preferred_element_type=_f32)
           + jnp.dot(xwhi * adhi_ref[...], Bhi, preferred_element_type=_f32))
    as_ref[...] = a_s
    ad_ref[...] = a_d
    bs = jnp.broadcast_to(jnp.max(a_s, axis=0, keepdims=True), (8, C))
    bd = jnp.broadcast_to(jnp.max(a_d, axis=0, keepdims=True), (8, C))

    @pl.when(i == 0)
    def _():
        ms_ref[...] = bs
        md_ref[...] = bd

    @pl.when(i > 0)
    def _():
        ms_ref[...] = jnp.maximum(ms_ref[...], bs)
        md_ref[...] = jnp.maximum(md_ref[...], bd)


def _tc_node_prologue(x, wlo, whi, aslo, ashi, adlo, adhi):
    w_spec = pl.BlockSpec((NFEAT, HW), lambda i: (0, 0))
    v_spec = pl.BlockSpec((1, HW), lambda i: (0, 0))
    return pl.pallas_call(
        _node_prologue_body,
        grid=(GN,),
        in_specs=[
            pl.BlockSpec((BN, NFEAT), lambda i: (i, 0)),
            w_spec, w_spec, v_spec, v_spec, v_spec, v_spec,
        ],
        out_specs=[
            pl.BlockSpec((2, BN, HW), lambda i: (0, i, 0)),
            pl.BlockSpec((BN, C), lambda i: (i, 0)),
            pl.BlockSpec((BN, C), lambda i: (i, 0)),
            pl.BlockSpec((8, C), lambda i: (0, 0)),
            pl.BlockSpec((8, C), lambda i: (0, 0)),
        ],
        out_shape=[
            jax.ShapeDtypeStruct((2, N, HW), _f32),
            jax.ShapeDtypeStruct((N, C), _f32),
            jax.ShapeDtypeStruct((N, C), _f32),
            jax.ShapeDtypeStruct((8, C), _f32),
            jax.ShapeDtypeStruct((8, C), _f32),
        ],
    )(x, wlo, whi, aslo, ashi, adlo, adhi)


def _edge_prologue_body(ea_ref, we_ref, atte_ref, ae_ref, me_ref):
    i = pl.program_id(0)
    ew = jnp.dot(ea_ref[...], we_ref[...], preferred_element_type=_f32)
    B = _bd(HC, C, 0)
    ae = jnp.dot(ew * atte_ref[...], B, preferred_element_type=_f32)
    col = lax.broadcasted_iota(jnp.int32, (BE, C), 1)
    ae = ae + jnp.where(col == H, 1.0, 0.0).astype(_f32)
    ae_ref[...] = ae
    bm = jnp.broadcast_to(jnp.max(ae, axis=0, keepdims=True), (8, C))

    @pl.when(i == 0)
    def _():
        me_ref[...] = bm

    @pl.when(i > 0)
    def _():
        me_ref[...] = jnp.maximum(me_ref[...], bm)


def _tc_edge_prologue(ea, we, attef):
    return pl.pallas_call(
        _edge_prologue_body,
        grid=(GE,),
        in_specs=[
            pl.BlockSpec((BE, C), lambda i: (i, 0)),
            pl.BlockSpec((C, HC), lambda i: (0, 0)),
            pl.BlockSpec((1, HC), lambda i: (0, 0)),
        ],
        out_specs=[
            pl.BlockSpec((BE, C), lambda i: (i, 0)),
            pl.BlockSpec((8, C), lambda i: (0, 0)),
        ],
        out_shape=[
            jax.ShapeDtypeStruct((E, C), _f32),
            jax.ShapeDtypeStruct((8, C), _f32),
        ],
    )(ea, we, attef)


def _sc_loopsum_body(rows_hbm, idx_hbm, out_hbm, idxb, rowsb, wbuf, acc):
    c = lax.axis_index("c")
    s = lax.axis_index("s")
    tid = s * NSC + c

    def zrow(i, _):
        wbuf[i, :] = jnp.zeros((C,), _f32)
        return 0

    lax.fori_loop(0, WB, zrow, 0)
    for k in range(NWB):
        pltpu.sync_copy(wbuf, acc.at[pl.ds(s * RPT + k * WB, WB)])
    plsc.subcore_barrier()

    def chunk(j, _):
        base = tid * EPT_A + j * CH
        pltpu.sync_copy(idx_hbm.at[pl.ds(base, CH)], idxb)
        pltpu.sync_copy(rows_hbm.at[pl.ds(base, CH)], rowsb)
        pltpu.sync_copy(rowsb, acc.at[idxb], add=True)
        return 0

    lax.fori_loop(0, NCH_A, chunk, 0)
    plsc.subcore_barrier()
    for k in range(NWB):
        sl = pl.ds(s * RPT + k * WB, WB)
        pltpu.sync_copy(acc.at[sl], wbuf)
        pltpu.sync_copy(wbuf, out_hbm.at[c, sl])


@functools.cache
def _get_sc_loopsum():
    return pl.kernel(
        _sc_loopsum_body,
        out_type=jax.ShapeDtypeStruct((NSC, N, C), _f32),
        mesh=plsc.VectorSubcoreMesh(core_axis_name="c", subcore_axis_name="s",
                                    num_cores=NSC, num_subcores=NTILE),
        scratch_types=[
            pltpu.VMEM((CH,), jnp.int32),
            pltpu.VMEM((CH, C), _f32),
            pltpu.VMEM((WB, C), _f32),
            pltpu.VMEM_SHARED((N, C), _f32),
        ],
        compiler_params=_SC_PARAMS,
    )


def _sc_edge_pass_body(has_edge, *refs):
    if has_edge:
        (src_hbm, dst_hbm, as_hbm, ad_hbm, ae_hbm, xw_hbm, ms_hbm, md_hbm,
         me_hbm, num_out, den_out, idxs, idxd, rs, rd, re, xwr, outr, exr,
         m1, m2, m3, wbuf, wbden, accn, accd, sem1, sem2, sem3) = refs
    else:
        (src_hbm, dst_hbm, as_hbm, ad_hbm, xw_hbm, ms_hbm, md_hbm,
         num_out, den_out, idxs, idxd, rs, rd, xwr, outr, exr,
         m1, m2, m3, wbuf, wbden, accn, accd, sem1, sem2, sem3) = refs
        re = None
    c = lax.axis_index("c")
    s = lax.axis_index("s")

    pltpu.sync_copy(ms_hbm, m1)
    pltpu.sync_copy(md_hbm, m2)
    S = m1[0, :] + m2[0, :]
    if has_edge:
        pltpu.sync_copy(me_hbm, m3)
        S = S + m3[0, :]
    S = jnp.maximum(S, 0.0)

    def zrow(i, _):
        for j in range(HH):
            wbuf[i, pl.ds(j * C, C)] = jnp.zeros((C,), _f32)
        wbden[i, :] = jnp.zeros((C,), _f32)
        return 0

    lax.fori_loop(0, WB, zrow, 0)
    for k in range(NWB):
        sl = pl.ds(s * RPT + k * WB, WB)
        pltpu.sync_copy(wbuf, accn.at[sl])
        pltpu.sync_copy(wbden, accd.at[sl])
    plsc.subcore_barrier()

    def chunk(j, _):
        base = s * EPT_B + j * CH
        pltpu.sync_copy(src_hbm.at[pl.ds(base, CH)], idxs)
        pltpu.sync_copy(dst_hbm.at[pl.ds(base, CH)], idxd)
        cp1 = pltpu.async_copy(as_hbm.at[idxs], rs, sem1)
        cp2 = pltpu.async_copy(ad_hbm.at[idxd], rd, sem2)
        cp3 = pltpu.async_copy(xw_hbm.at[c].at[idxs], xwr, sem3)
        if has_edge:
            pltpu.sync_copy(ae_hbm.at[pl.ds(base, CH)], re)
        cp1.wait()
        cp2.wait()
        cp3.wait()

        def edge(e, _):
            a = rs[e, :] + rd[e, :]
            if has_edge:
                a = a + re[e, :]
            a = jnp.where(a >= 0.0, a, 0.2 * a)
            ex = jnp.exp(a - S)
            exr[e, :] = ex
            idxe = jnp.full((C,), e, jnp.int32)
            for h in range(HH):
                idxh = jnp.full((C,), h, jnp.int32) + c * HH
                b = plsc.load_gather(exr, [idxe, idxh])
                outr[e, pl.ds(h * C, C)] = xwr[e, pl.ds(h * C, C)] * b
            return 0

        lax.fori_loop(0, CH, edge, 0)
        pltpu.sync_copy(outr, accn.at[idxd], add=True)

        @pl.when(c == 0)
        def _():
            pltpu.sync_copy(exr, accd.at[idxd], add=True)

        return 0

    lax.fori_loop(0, NCH_B, chunk, 0)
    plsc.subcore_barrier()
    for k in range(NWB):
        sl = pl.ds(s * RPT + k * WB, WB)
        pltpu.sync_copy(accn.at[sl], wbuf)
        pltpu.sync_copy(wbuf, num_out.at[c, sl])

    @pl.when(c == 0)
    def _():
        for k in range(NWB):
            sl = pl.ds(s * RPT + k * WB, WB)
            pltpu.sync_copy(accd.at[sl], wbden)
            pltpu.sync_copy(wbden, den_out.at[sl])


@functools.cache
def _make_sc_edge_pass(has_edge):
    scratch = [
        pltpu.VMEM((CH,), jnp.int32),
        pltpu.VMEM((CH,), jnp.int32),
        pltpu.VMEM((CH, C), _f32),
        pltpu.VMEM((CH, C), _f32),
    ]
    if has_edge:
        scratch.append(pltpu.VMEM((CH, C), _f32))
    scratch += [
        pltpu.VMEM((CH, HW), _f32),
        pltpu.VMEM((CH, HW), _f32),
        pltpu.VMEM((CH, C), _f32),
        pltpu.VMEM((8, C), _f32),
        pltpu.VMEM((8, C), _f32),
        pltpu.VMEM((8, C), _f32),
        pltpu.VMEM((WB, HW), _f32),
        pltpu.VMEM((WB, C), _f32),
        pltpu.VMEM_SHARED((N, HW), _f32),
        pltpu.VMEM_SHARED((N, C), _f32),
        pltpu.SemaphoreType.DMA,
        pltpu.SemaphoreType.DMA,
        pltpu.SemaphoreType.DMA,
    ]
    return pl.kernel(
        functools.partial(_sc_edge_pass_body, has_edge),
        out_type=(
            jax.ShapeDtypeStruct((NSC, N, HW), _f32),
            jax.ShapeDtypeStruct((N, C), _f32),
        ),
        mesh=plsc.VectorSubcoreMesh(core_axis_name="c", subcore_axis_name="s",
                                    num_cores=NSC, num_subcores=NTILE),
        scratch_types=scratch,
        compiler_params=_SC_PARAMS,
    )


def _combine1_body(nm_ref, dn_ref, l0_ref, l1_ref,
                   as_ref, ad_ref, xw_ref, ms_ref, md_ref, me_ref,
                   b1lo_ref, b1hi_ref, wll_ref, wlh_ref, whl_ref, whh_ref,
                   as2lo_ref, as2hi_ref, ad2lo_ref, ad2hi_ref,
                   xw2_ref, as2_ref, ad2_ref, ms2_ref, md2_ref):
    i = pl.program_id(0)
    den = dn_ref[...]
    ls = l0_ref[...] + l1_ref[...]
    onehot12 = jnp.where(
        lax.broadcasted_iota(jnp.int32, (1, C), 1) == H, 1.0, 0.0).astype(_f32)
    cnt = jnp.sum(ls * onehot12, axis=1, keepdims=True)
    lae = ls / jnp.maximum(cnt, 1.0)
    S = jnp.maximum(ms_ref[0:1, :] + md_ref[0:1, :] + me_ref[0:1, :], 0.0)
    al = as_ref[...] + ad_ref[...] + lae
    al = jnp.where(al >= 0.0, al, 0.2 * al)
    exl = jnp.exp(al - S)
    Rlo = _bd_t(C, HW, 0)
    Rhi = _bd_t(C, HW, HH)
    den_t = den + exl
    dinv = 1.0 / (den_t + 1e-16)
    hs = []
    for half, R, b1 in ((0, Rlo, b1lo_ref), (1, Rhi, b1hi_ref)):
        exb = jnp.dot(exl, R, preferred_element_type=_f32)
        num_t = nm_ref[half] + xw_ref[half] * exb
        dinvb = jnp.dot(dinv, R, preferred_element_type=_f32)
        hv = num_t * dinvb + b1[...]
        hs.append(jnp.where(hv > 0.0, hv, jnp.exp(hv) - 1.0))
    xw2lo = (jnp.dot(hs[0], wll_ref[...], preferred_element_type=_f32)
             + jnp.dot(hs[1], whl_ref[...], preferred_element_type=_f32))
    xw2hi = (jnp.dot(hs[0], wlh_ref[...], preferred_element_type=_f32)
             + jnp.dot(hs[1], whh_ref[...], preferred_element_type=_f32))
    xw2_ref[0] = xw2lo
    xw2_ref[1] = xw2hi
    Blo = _bd(HW, C, 0)
    Bhi = _bd(HW, C, HH)
    a_s2 = (jnp.dot(xw2lo * as2lo_ref[...], Blo, preferred_element_type=_f32)
            + jnp.dot(xw2hi * as2hi_ref[...], Bhi,
                      preferred_element_type=_f32))
    a_d2 = (jnp.dot(xw2lo * ad2lo_ref[...], Blo, preferred_element_type=_f32)
            + jnp.dot(xw2hi * ad2hi_ref[...], Bhi,
                      preferred_element_type=_f32))
    as2_ref[...] = a_s2
    ad2_ref[...] = a_d2
    bs = jnp.broadcast_to(jnp.max(a_s2, axis=0, keepdims=True), (8, C))
    bd = jnp.broadcast_to(jnp.max(a_d2, axis=0, keepdims=True), (8, C))

    @pl.when(i == 0)
    def _():
        ms2_ref[...] = bs
        md2_ref[...] = bd

    @pl.when(i > 0)
    def _():
        ms2_ref[...] = jnp.maximum(ms2_ref[...], bs)
        md2_ref[...] = jnp.maximum(md2_ref[...], bd)


def _tc_combine1(nm, dn, l0, l1, asrc, adst, xw, ms, md, me,
                 b1lo, b1hi, wll, wlh, whl, whh,
                 as2lo, as2hi, ad2lo, ad2hi):
    spec_cat = pl.BlockSpec((2, BN, HW), lambda i: (0, i, 0))
    spec_n16 = pl.BlockSpec((BN, C), lambda i: (i, 0))
    spec_m = pl.BlockSpec((8, C), lambda i: (0, 0))
    spec_v = pl.BlockSpec((1, HW), lambda i: (0, 0))
    spec_w = pl.BlockSpec((HW, HW), lambda i: (0, 0))
    return pl.pallas_call(
        _combine1_body,
        grid=(GN,),
        in_specs=[
            spec_cat, spec_n16, spec_n16, spec_n16,
            spec_n16, spec_n16, spec_cat,
            spec_m, spec_m, spec_m,
            spec_v, spec_v, spec_w, spec_w, spec_w, spec_w,
            spec_v, spec_v, spec_v, spec_v,
        ],
        out_specs=[spec_cat, spec_n16, spec_n16, spec_m, spec_m],
        out_shape=[
            jax.ShapeDtypeStruct((2, N, HW), _f32),
            jax.ShapeDtypeStruct((N, C), _f32),
            jax.ShapeDtypeStruct((N, C), _f32),
            jax.ShapeDtypeStruct((8, C), _f32),
            jax.ShapeDtypeStruct((8, C), _f32),
        ],
    )(nm, dn, l0, l1, asrc, adst, xw, ms, md, me,
      b1lo, b1hi, wll, wlh, whl, whh, as2lo, as2hi, ad2lo, ad2hi)


def _combine2_body(nm_ref, dn_ref, as_ref, ad_ref, xw_ref,
                   ms_ref, md_ref, b2_ref, h2_ref, lp_ref):
    den = dn_ref[...]
    S = jnp.maximum(ms_ref[0:1, :] + md_ref[0:1, :], 0.0)
    al = as_ref[...] + ad_ref[...]
    al = jnp.where(al >= 0.0, al, 0.2 * al)
    exl = jnp.exp(al - S)
    den_t = den + exl
    dinv = 1.0 / (den_t + 1e-16)
    rr = lax.broadcasted_iota(jnp.int32, (HW, C), 0)
    cc = lax.broadcasted_iota(jnp.int32, (HW, C), 1)
    Rm = jnp.where(rr % C == cc, 1.0 / H, 0.0).astype(_f32)
    h2 = b2_ref[...]
    for half, R in ((0, _bd_t(C, HW, 0)), (1, _bd_t(C, HW, HH))):
        exb = jnp.dot(exl, R, preferred_element_type=_f32)
        num_t = nm_ref[half] + xw_ref[half] * exb
        dinvb = jnp.dot(dinv, R, preferred_element_type=_f32)
        h2 = h2 + jnp.dot(num_t * dinvb, Rm, preferred_element_type=_f32)
    h2_ref[...] = h2
    m = jnp.max(h2, axis=1, keepdims=True)
    z = h2 - m
    lse = jnp.log(jnp.sum(jnp.exp(z), axis=1, keepdims=True))
    lp_ref[...] = z - lse


def _tc_combine2(nm, dn, asrc, adst, xw, ms, md, b2r):
    spec_cat = pl.BlockSpec((2, BN, HW), lambda i: (0, i, 0))
    spec_n16 = pl.BlockSpec((BN, C), lambda i: (i, 0))
    spec_m = pl.BlockSpec((8, C), lambda i: (0, 0))
    return pl.pallas_call(
        _combine2_body,
        grid=(GN,),
        in_specs=[
            spec_cat, spec_n16, spec_n16, spec_n16, spec_cat,
            spec_m, spec_m,
            pl.BlockSpec((1, C), lambda i: (0, 0)),
        ],
        out_specs=[spec_n16, spec_n16],
        out_shape=[
            jax.ShapeDtypeStruct((N, C), _f32),
            jax.ShapeDtypeStruct((N, C), _f32),
        ],
    )(nm, dn, asrc, adst, xw, ms, md, b2r)


def kernel(x, edge_index, edge_attr, W1, att_src1, att_dst1, W_edge1,
           att_edge1, bias1, W2, att_src2, att_dst2, bias2):
    src = edge_index[0]
    dst = edge_index[1]
    w1lo, w1hi = W1[:, :HW], W1[:, HW:]
    as1lo = att_src1[:HH].reshape(1, HW)
    as1hi = att_src1[HH:].reshape(1, HW)
    ad1lo = att_dst1[:HH].reshape(1, HW)
    ad1hi = att_dst1[HH:].reshape(1, HW)
    attef1 = att_edge1.reshape(1, HC)
    as2lo = att_src2[:HH].reshape(1, HW)
    as2hi = att_src2[HH:].reshape(1, HW)
    ad2lo = att_dst2[:HH].reshape(1, HW)
    ad2hi = att_dst2[HH:].reshape(1, HW)
    wll, wlh = W2[:HW, :HW], W2[:HW, HW:]
    whl, whh = W2[HW:, :HW], W2[HW:, HW:]
    b1lo = bias1[:HW].reshape(1, HW)
    b1hi = bias1[HW:].reshape(1, HW)

    xw1, asrc1, adst1, ms1, md1 = _tc_node_prologue(
        x, w1lo, w1hi, as1lo, as1hi, ad1lo, ad1hi)
    ae1, me1 = _tc_edge_prologue(edge_attr, W_edge1, attef1)
    loops = _get_sc_loopsum()(ae1, dst)
    num1, den1 = _make_sc_edge_pass(True)(src, dst, asrc1, adst1, ae1, xw1,
                                          ms1, md1, me1)
    xw2, asrc2, adst2, ms2, md2 = _tc_combine1(
        num1, den1, loops[0], loops[1], asrc1, adst1, xw1, ms1, md1, me1,
        b1lo, b1hi, wll, wlh, whl, whh, as2lo, as2hi, ad2lo, ad2hi)
    num2, den2 = _make_sc_edge_pass(False)(src, dst, asrc2, adst2, xw2,
                                           ms2, md2)
    h2, lp = _tc_combine2(num2, den2, asrc2, adst2, xw2, ms2, md2,
                          bias2.reshape(1, C))
    return (h2, lp)


# speedup vs baseline: 31.6330x; 1.2381x over previous
import functools

import jax
import jax.numpy as jnp
from jax import lax
from jax.experimental import pallas as pl
from jax.experimental.pallas import tpu as pltpu
from jax.experimental.pallas import tpu_sc as plsc

N = 10000
E = 320000
NFEAT = 128
H = 12
C = 16
HC = H * C
HH = H // 2
HW = HH * C

BN = 400
GN = N // BN
BE = 3200
GE = E // BE

NSC = 2
NTILE = 16
NW = NSC * NTILE
CH = 80
EPT_A = E // NW
NCH_A = EPT_A // CH
EPT_B = E // NTILE
NCH_B = EPT_B // CH
RPT = N // NTILE
WB = 125
NWB = RPT // WB

_f32 = jnp.float32

_SC_PARAMS = pltpu.CompilerParams(
    use_tc_tiling_on_sc=False, needs_layout_passes=False)


def _bd(rows, cols, shift):
    r = lax.broadcasted_iota(jnp.int32, (rows, cols), 0)
    c = lax.broadcasted_iota(jnp.int32, (rows, cols), 1)
    return (r // C + shift == c).astype(_f32)


def _bd_t(rows, cols, shift):
    r = lax.broadcasted_iota(jnp.int32, (rows, cols), 0)
    c = lax.broadcasted_iota(jnp.int32, (rows, cols), 1)
    return (c // C + shift == r).astype(_f32)


def _node_prologue_body(x_ref, wlo_ref, whi_ref, aslo_ref, ashi_ref,
                        adlo_ref, adhi_ref,
                        xw_ref, as_ref, ad_ref, ms_ref, md_ref):
    i = pl.program_id(0)
    xb = x_ref[...]
    xwlo = jnp.dot(xb, wlo_ref[...], preferred_element_type=_f32)
    xwhi = jnp.dot(xb, whi_ref[...], preferred_element_type=_f32)
    xw_ref[0] = xwlo
    xw_ref[1] = xwhi
    Blo = _bd(HW, C, 0)
    Bhi = _bd(HW, C, HH)
    a_s = (jnp.dot(xwlo * aslo_ref[...], Blo, preferred_element_type=_f32)
           + jnp.dot(xwhi * ashi_ref[...], Bhi, preferred_element_type=_f32))
    a_d = (jnp.dot(xwlo * adlo_ref[...], Blo, preferred_element_type=_f32)
           + jnp.dot(xwhi * adhi_ref[...], Bhi, preferred_element_type=_f32))
    as_ref[...] = a_s
    ad_ref[...] = a_d
    bs = jnp.broadcast_to(jnp.max(a_s, axis=0, keepdims=True), (8, C))
    bd = jnp.broadcast_to(jnp.max(a_d, axis=0, keepdims=True), (8, C))

    @pl.when(i == 0)
    def _():
        ms_ref[...] = bs
        md_ref[...] = bd

    @pl.when(i > 0)
    def _():
        ms_ref[...] = jnp.maximum(ms_ref[...], bs)
        md_ref[...] = jnp.maximum(md_ref[...], bd)


def _tc_node_prologue(x, wlo, whi, aslo, ashi, adlo, adhi):
    w_spec = pl.BlockSpec((NFEAT, HW), lambda i: (0, 0))
    v_spec = pl.BlockSpec((1, HW), lambda i: (0, 0))
    return pl.pallas_call(
        _node_prologue_body,
        grid=(GN,),
        in_specs=[
            pl.BlockSpec((BN, NFEAT), lambda i: (i, 0)),
            w_spec, w_spec, v_spec, v_spec, v_spec, v_spec,
        ],
        out_specs=[
            pl.BlockSpec((2, BN, HW), lambda i: (0, i, 0)),
            pl.BlockSpec((BN, C), lambda i: (i, 0)),
            pl.BlockSpec((BN, C), lambda i: (i, 0)),
            pl.BlockSpec((8, C), lambda i: (0, 0)),
            pl.BlockSpec((8, C), lambda i: (0, 0)),
        ],
        out_shape=[
            jax.ShapeDtypeStruct((2, N, HW), _f32),
            jax.ShapeDtypeStruct((N, C), _f32),
            jax.ShapeDtypeStruct((N, C), _f32),
            jax.ShapeDtypeStruct((8, C), _f32),
            jax.ShapeDtypeStruct((8, C), _f32),
        ],
    )(x, wlo, whi, aslo, ashi, adlo, adhi)


def _edge_prologue_body(ea_ref, we_ref, atte_ref, ae_ref, me_ref):
    i = pl.program_id(0)
    ew = jnp.dot(ea_ref[...], we_ref[...], preferred_element_type=_f32)
    B = _bd(HC, C, 0)
    ae = jnp.dot(ew * atte_ref[...], B, preferred_element_type=_f32)
    col = lax.broadcasted_iota(jnp.int32, (BE, C), 1)
    ae = ae + jnp.where(col == H, 1.0, 0.0).astype(_f32)
    ae_ref[...] = ae
    bm = jnp.broadcast_to(jnp.max(ae, axis=0, keepdims=True), (8, C))

    @pl.when(i == 0)
    def _():
        me_ref[...] = bm

    @pl.when(i > 0)
    def _():
        me_ref[...] = jnp.maximum(me_ref[...], bm)


def _tc_edge_prologue(ea, we, attef):
    return pl.pallas_call(
        _edge_prologue_body,
        grid=(GE,),
        in_specs=[
            pl.BlockSpec((BE, C), lambda i: (i, 0)),
            pl.BlockSpec((C, HC), lambda i: (0, 0)),
            pl.BlockSpec((1, HC), lambda i: (0, 0)),
        ],
        out_specs=[
            pl.BlockSpec((BE, C), lambda i: (i, 0)),
            pl.BlockSpec((8, C), lambda i: (0, 0)),
        ],
        out_shape=[
            jax.ShapeDtypeStruct((E, C), _f32),
            jax.ShapeDtypeStruct((8, C), _f32),
        ],
    )(ea, we, attef)


def _sc_loopsum_body(rows_hbm, idx_hbm, out_hbm, idxb, rowsb, wbuf, acc):
    c = lax.axis_index("c")
    s = lax.axis_index("s")
    tid = s * NSC + c

    def zrow(i, _):
        wbuf[i, :] = jnp.zeros((C,), _f32)
        return 0

    lax.fori_loop(0, WB, zrow, 0)
    for k in range(NWB):
        pltpu.sync_copy(wbuf, acc.at[pl.ds(s * RPT + k * WB, WB)])
    plsc.subcore_barrier()

    def chunk(j, _):
        base = tid * EPT_A + j * CH
        pltpu.sync_copy(idx_hbm.at[pl.ds(base, CH)], idxb)
        pltpu.sync_copy(rows_hbm.at[pl.ds(base, CH)], rowsb)
        pltpu.sync_copy(rowsb, acc.at[idxb], add=True)
        return 0

    lax.fori_loop(0, NCH_A, chunk, 0)
    plsc.subcore_barrier()
    for k in range(NWB):
        sl = pl.ds(s * RPT + k * WB, WB)
        pltpu.sync_copy(acc.at[sl], wbuf)
        pltpu.sync_copy(wbuf, out_hbm.at[c, sl])


@functools.cache
def _get_sc_loopsum():
    return pl.kernel(
        _sc_loopsum_body,
        out_type=jax.ShapeDtypeStruct((NSC, N, C), _f32),
        mesh=plsc.VectorSubcoreMesh(core_axis_name="c", subcore_axis_name="s",
                                    num_cores=NSC, num_subcores=NTILE),
        scratch_types=[
            pltpu.VMEM((CH,), jnp.int32),
            pltpu.VMEM((CH, C), _f32),
            pltpu.VMEM((WB, C), _f32),
            pltpu.VMEM_SHARED((N, C), _f32),
        ],
        compiler_params=_SC_PARAMS,
    )


def _sc_edge_pass_body(has_edge, *refs):
    if has_edge:
        (src_hbm, dst_hbm, as_hbm, ad_hbm, ae_hbm, xw_hbm, ms_hbm, md_hbm,
         me_hbm, num_out, den_out,
         idxs0, idxs1, idxd0, idxd1, idxc0, idxc1,
         rs0, rs1, rd0, rd1, re0, re1, xwr0, xwr1, outr0, outr1, exr0, exr1,
         m1, m2, m3, wbuf, wbden, accn, accd,
         gsem0, gsem1, ssem0, ssem1) = refs
        re_ = (re0, re1)
    else:
        (src_hbm, dst_hbm, as_hbm, ad_hbm, xw_hbm, ms_hbm, md_hbm,
         num_out, den_out,
         idxs0, idxs1, idxd0, idxd1, idxc0, idxc1,
         rs0, rs1, rd0, rd1, xwr0, xwr1, outr0, outr1, exr0, exr1,
         m1, m2, m3, wbuf, wbden, accn, accd,
         gsem0, gsem1, ssem0, ssem1) = refs
        re_ = (None, None)
    idxs_ = (idxs0, idxs1)
    idxd_ = (idxd0, idxd1)
    idxc_ = (idxc0, idxc1)
    rs_ = (rs0, rs1)
    rd_ = (rd0, rd1)
    xwr_ = (xwr0, xwr1)
    outr_ = (outr0, outr1)
    exr_ = (exr0, exr1)
    gsem_ = (gsem0, gsem1)
    ssem_ = (ssem0, ssem1)
    c = lax.axis_index("c")
    s = lax.axis_index("s")

    pltpu.sync_copy(ms_hbm, m1)
    pltpu.sync_copy(md_hbm, m2)
    S = m1[0, :] + m2[0, :]
    if has_edge:
        pltpu.sync_copy(me_hbm, m3)
        S = S + m3[0, :]
    S = jnp.maximum(S, 0.0)
    idxh_ = [jnp.full((C,), h, jnp.int32) + c * HH for h in range(HH)]

    def zrow(i, _):
        for j in range(HH):
            wbuf[i, pl.ds(j * C, C)] = jnp.zeros((C,), _f32)
        wbden[i, :] = jnp.zeros((C,), _f32)
        return 0

    lax.fori_loop(0, WB, zrow, 0)
    for k in range(NWB):
        sl = pl.ds(s * RPT + k * WB, WB)
        pltpu.sync_copy(wbuf, accn.at[sl])
        pltpu.sync_copy(wbden, accd.at[sl])
    plsc.subcore_barrier()

    def start_gathers(j, b):
        base = s * EPT_B + j * CH
        pltpu.sync_copy(src_hbm.at[pl.ds(base, CH)], idxs_[b])
        pltpu.sync_copy(dst_hbm.at[pl.ds(base, CH)], idxd_[b])
        pltpu.async_copy(as_hbm.at[idxs_[b]], rs_[b], gsem_[b])
        pltpu.async_copy(ad_hbm.at[idxd_[b]], rd_[b], gsem_[b])
        pltpu.async_copy(xw_hbm.at[c].at[idxs_[b]], xwr_[b], gsem_[b])
        if has_edge:
            pltpu.async_copy(ae_hbm.at[pl.ds(base, CH)], re_[b], gsem_[b])

    def wait_gathers(b):
        pltpu.make_async_copy(as_hbm.at[idxs_[b]], rs_[b], gsem_[b]).wait()
        pltpu.make_async_copy(ad_hbm.at[idxd_[b]], rd_[b], gsem_[b]).wait()
        pltpu.make_async_copy(
            xw_hbm.at[c].at[idxs_[b]], xwr_[b], gsem_[b]).wait()
        if has_edge:
            pltpu.make_async_copy(
                ae_hbm.at[pl.ds(0, CH)], re_[b], gsem_[b]).wait()

    def wait_scatters(b):
        pltpu.make_async_copy(
            outr_[b], accn.at[idxc_[b]], ssem_[b]).wait()

        @pl.when(c == 0)
        def _():
            pltpu.make_async_copy(
                exr_[b], accd.at[idxc_[b]], ssem_[b]).wait()

    start_gathers(0, 0)
    start_gathers(1, 1)

    def outer(jo, _):
        for b in (0, 1):
            j = 2 * jo + b
            wait_gathers(b)

            @pl.when(jo > 0)
            def _():
                wait_scatters(b)

            rs, rd, re = rs_[b], rd_[b], re_[b]
            xwr, outr, exr = xwr_[b], outr_[b], exr_[b]

            def edge(e, _):
                a = rs[e, :] + rd[e, :]
                if has_edge:
                    a = a + re[e, :]
                a = jnp.where(a >= 0.0, a, 0.2 * a)
                ex = jnp.exp(a - S)
                exr[e, :] = ex
                idxe = jnp.full((C,), e, jnp.int32)
                for h in range(HH):
                    bc = plsc.load_gather(exr, [idxe, idxh_[h]])
                    outr[e, pl.ds(h * C, C)] = xwr[e, pl.ds(h * C, C)] * bc
                return 0

            lax.fori_loop(0, CH, edge, 0, unroll=4)
            for k in range(CH // C):
                idxc_[b][pl.ds(k * C, C)] = idxd_[b][pl.ds(k * C, C)]
            pltpu.async_copy(outr, accn.at[idxc_[b]], ssem_[b], add=True)

            @pl.when(c == 0)
            def _():
                pltpu.async_copy(exr, accd.at[idxc_[b]], ssem_[b], add=True)

            @pl.when(jo < NCH_B // 2 - 1)
            def _():
                start_gathers(j + 2, b)

        return 0

    lax.fori_loop(0, NCH_B // 2, outer, 0)
    wait_scatters(0)
    wait_scatters(1)
    plsc.subcore_barrier()
    for k in range(NWB):
        sl = pl.ds(s * RPT + k * WB, WB)
        pltpu.sync_copy(accn.at[sl], wbuf)
        pltpu.sync_copy(wbuf, num_out.at[c, sl])

    @pl.when(c == 0)
    def _():
        for k in range(NWB):
            sl = pl.ds(s * RPT + k * WB, WB)
            pltpu.sync_copy(accd.at[sl], wbden)
            pltpu.sync_copy(wbden, den_out.at[sl])


@functools.cache
def _make_sc_edge_pass(has_edge):
    scratch = [
        pltpu.VMEM((CH,), jnp.int32),
        pltpu.VMEM((CH,), jnp.int32),
        pltpu.VMEM((CH,), jnp.int32),
        pltpu.VMEM((CH,), jnp.int32),
        pltpu.VMEM((CH,), jnp.int32),
        pltpu.VMEM((CH,), jnp.int32),
        pltpu.VMEM((CH, C), _f32),
        pltpu.VMEM((CH, C), _f32),
        pltpu.VMEM((CH, C), _f32),
        pltpu.VMEM((CH, C), _f32),
    ]
    if has_edge:
        scratch += [pltpu.VMEM((CH, C), _f32),
                    pltpu.VMEM((CH, C), _f32)]
    scratch += [
        pltpu.VMEM((CH, HW), _f32),
        pltpu.VMEM((CH, HW), _f32),
        pltpu.VMEM((CH, HW), _f32),
        pltpu.VMEM((CH, HW), _f32),
        pltpu.VMEM((CH, C), _f32),
        pltpu.VMEM((CH, C), _f32),
        pltpu.VMEM((8, C), _f32),
        pltpu.VMEM((8, C), _f32),
        pltpu.VMEM((8, C), _f32),
        pltpu.VMEM((WB, HW), _f32),
        pltpu.VMEM((WB, C), _f32),
        pltpu.VMEM_SHARED((N, HW), _f32),
        pltpu.VMEM_SHARED((N, C), _f32),
        pltpu.SemaphoreType.DMA,
        pltpu.SemaphoreType.DMA,
        pltpu.SemaphoreType.DMA,
        pltpu.SemaphoreType.DMA,
    ]
    return pl.kernel(
        functools.partial(_sc_edge_pass_body, has_edge),
        out_type=(
            jax.ShapeDtypeStruct((NSC, N, HW), _f32),
            jax.ShapeDtypeStruct((N, C), _f32),
        ),
        mesh=plsc.VectorSubcoreMesh(core_axis_name="c", subcore_axis_name="s",
                                    num_cores=NSC, num_subcores=NTILE),
        scratch_types=scratch,
        compiler_params=_SC_PARAMS,
    )


def _combine1_body(nm_ref, dn_ref, l0_ref, l1_ref,
                   as_ref, ad_ref, xw_ref, ms_ref, md_ref, me_ref,
                   b1lo_ref, b1hi_ref, wll_ref, wlh_ref, whl_ref, whh_ref,
                   as2lo_ref, as2hi_ref, ad2lo_ref, ad2hi_ref,
                   xw2_ref, as2_ref, ad2_ref, ms2_ref, md2_ref):
    i = pl.program_id(0)
    den = dn_ref[...]
    ls = l0_ref[...] + l1_ref[...]
    onehot12 = jnp.where(
        lax.broadcasted_iota(jnp.int32, (1, C), 1) == H, 1.0, 0.0).astype(_f32)
    cnt = jnp.sum(ls * onehot12, axis=1, keepdims=True)
    lae = ls / jnp.maximum(cnt, 1.0)
    S = jnp.maximum(ms_ref[0:1, :] + md_ref[0:1, :] + me_ref[0:1, :], 0.0)
    al = as_ref[...] + ad_ref[...] + lae
    al = jnp.where(al >= 0.0, al, 0.2 * al)
    exl = jnp.exp(al - S)
    Rlo = _bd_t(C, HW, 0)
    Rhi = _bd_t(C, HW, HH)
    den_t = den + exl
    dinv = 1.0 / (den_t + 1e-16)
    hs = []
    for half, R, b1 in ((0, Rlo, b1lo_ref), (1, Rhi, b1hi_ref)):
        exb = jnp.dot(exl, R, preferred_element_type=_f32)
        num_t = nm_ref[half] + xw_ref[half] * exb
        dinvb = jnp.dot(dinv, R, preferred_element_type=_f32)
        hv = num_t * dinvb + b1[...]
        hs.append(jnp.where(hv > 0.0, hv, jnp.exp(hv) - 1.0))
    xw2lo = (jnp.dot(hs[0], wll_ref[...], preferred_element_type=_f32)
             + jnp.dot(hs[1], whl_ref[...], preferred_element_type=_f32))
    xw2hi = (jnp.dot(hs[0], wlh_ref[...], preferred_element_type=_f32)
             + jnp.dot(hs[1], whh_ref[...], preferred_element_type=_f32))
    xw2_ref[0] = xw2lo
    xw2_ref[1] = xw2hi
    Blo = _bd(HW, C, 0)
    Bhi = _bd(HW, C, HH)
    a_s2 = (jnp.dot(xw2lo * as2lo_ref[...], Blo, preferred_element_type=_f32)
            + jnp.dot(xw2hi * as2hi_ref[...], Bhi,
                      preferred_element_type=_f32))
    a_d2 = (jnp.dot(xw2lo * ad2lo_ref[...], Blo, preferred_element_type=_f32)
            + jnp.dot(xw2hi * ad2hi_ref[...], Bhi,
                      preferred_element_type=_f32))
    as2_ref[...] = a_s2
    ad2_ref[...] = a_d2
    bs = jnp.broadcast_to(jnp.max(a_s2, axis=0, keepdims=True), (8, C))
    bd = jnp.broadcast_to(jnp.max(a_d2, axis=0, keepdims=True), (8, C))

    @pl.when(i == 0)
    def _():
        ms2_ref[...] = bs
        md2_ref[...] = bd

    @pl.when(i > 0)
    def _():
        ms2_ref[...] = jnp.maximum(ms2_ref[...], bs)
        md2_ref[...] = jnp.maximum(md2_ref[...], bd)


def _tc_combine1(nm, dn, l0, l1, asrc, adst, xw, ms, md, me,
                 b1lo, b1hi, wll, wlh, whl, whh,
                 as2lo, as2hi, ad2lo, ad2hi):
    spec_cat = pl.BlockSpec((2, BN, HW), lambda i: (0, i, 0))
    spec_n16 = pl.BlockSpec((BN, C), lambda i: (i, 0))
    spec_m = pl.BlockSpec((8, C), lambda i: (0, 0))
    spec_v = pl.BlockSpec((1, HW), lambda i: (0, 0))
    spec_w = pl.BlockSpec((HW, HW), lambda i: (0, 0))
    return pl.pallas_call(
        _combine1_body,
        grid=(GN,),
        in_specs=[
            spec_cat, spec_n16, spec_n16, spec_n16,
            spec_n16, spec_n16, spec_cat,
            spec_m, spec_m, spec_m,
            spec_v, spec_v, spec_w, spec_w, spec_w, spec_w,
            spec_v, spec_v, spec_v, spec_v,
        ],
        out_specs=[spec_cat, spec_n16, spec_n16, spec_m, spec_m],
        out_shape=[
            jax.ShapeDtypeStruct((2, N, HW), _f32),
            jax.ShapeDtypeStruct((N, C), _f32),
            jax.ShapeDtypeStruct((N, C), _f32),
            jax.ShapeDtypeStruct((8, C), _f32),
            jax.ShapeDtypeStruct((8, C), _f32),
        ],
    )(nm, dn, l0, l1, asrc, adst, xw, ms, md, me,
      b1lo, b1hi, wll, wlh, whl, whh, as2lo, as2hi, ad2lo, ad2hi)


def _combine2_body(nm_ref, dn_ref, as_ref, ad_ref, xw_ref,
                   ms_ref, md_ref, b2_ref, h2_ref, lp_ref):
    den = dn_ref[...]
    S = jnp.maximum(ms_ref[0:1, :] + md_ref[0:1, :], 0.0)
    al = as_ref[...] + ad_ref[...]
    al = jnp.where(al >= 0.0, al, 0.2 * al)
    exl = jnp.exp(al - S)
    den_t = den + exl
    dinv = 1.0 / (den_t + 1e-16)
    rr = lax.broadcasted_iota(jnp.int32, (HW, C), 0)
    cc = lax.broadcasted_iota(jnp.int32, (HW, C), 1)
    Rm = jnp.where(rr % C == cc, 1.0 / H, 0.0).astype(_f32)
    h2 = b2_ref[...]
    for half, R in ((0, _bd_t(C, HW, 0)), (1, _bd_t(C, HW, HH))):
        exb = jnp.dot(exl, R, preferred_element_type=_f32)
        num_t = nm_ref[half] + xw_ref[half] * exb
        dinvb = jnp.dot(dinv, R, preferred_element_type=_f32)
        h2 = h2 + jnp.dot(num_t * dinvb, Rm, preferred_element_type=_f32)
    h2_ref[...] = h2
    m = jnp.max(h2, axis=1, keepdims=True)
    z = h2 - m
    lse = jnp.log(jnp.sum(jnp.exp(z), axis=1, keepdims=True))
    lp_ref[...] = z - lse


def _tc_combine2(nm, dn, asrc, adst, xw, ms, md, b2r):
    spec_cat = pl.BlockSpec((2, BN, HW), lambda i: (0, i, 0))
    spec_n16 = pl.BlockSpec((BN, C), lambda i: (i, 0))
    spec_m = pl.BlockSpec((8, C), lambda i: (0, 0))
    return pl.pallas_call(
        _combine2_body,
        grid=(GN,),
        in_specs=[
            spec_cat, spec_n16, spec_n16, spec_n16, spec_cat,
            spec_m, spec_m,
            pl.BlockSpec((1, C), lambda i: (0, 0)),
        ],
        out_specs=[spec_n16, spec_n16],
        out_shape=[
            jax.ShapeDtypeStruct((N, C), _f32),
            jax.ShapeDtypeStruct((N, C), _f32),
        ],
    )(nm, dn, asrc, adst, xw, ms, md, b2r)


def kernel(x, edge_index, edge_attr, W1, att_src1, att_dst1, W_edge1,
           att_edge1, bias1, W2, att_src2, att_dst2, bias2):
    src = edge_index[0]
    dst = edge_index[1]
    w1lo, w1hi = W1[:, :HW], W1[:, HW:]
    as1lo = att_src1[:HH].reshape(1, HW)
    as1hi = att_src1[HH:].reshape(1, HW)
    ad1lo = att_dst1[:HH].reshape(1, HW)
    ad1hi = att_dst1[HH:].reshape(1, HW)
    attef1 = att_edge1.reshape(1, HC)
    as2lo = att_src2[:HH].reshape(1, HW)
    as2hi = att_src2[HH:].reshape(1, HW)
    ad2lo = att_dst2[:HH].reshape(1, HW)
    ad2hi = att_dst2[HH:].reshape(1, HW)
    wll, wlh = W2[:HW, :HW], W2[:HW, HW:]
    whl, whh = W2[HW:, :HW], W2[HW:, HW:]
    b1lo = bias1[:HW].reshape(1, HW)
    b1hi = bias1[HW:].reshape(1, HW)

    xw1, asrc1, adst1, ms1, md1 = _tc_node_prologue(
        x, w1lo, w1hi, as1lo, as1hi, ad1lo, ad1hi)
    ae1, me1 = _tc_edge_prologue(edge_attr, W_edge1, attef1)
    loops = _get_sc_loopsum()(ae1, dst)
    num1, den1 = _make_sc_edge_pass(True)(src, dst, asrc1, adst1, ae1, xw1,
                                          ms1, md1, me1)
    xw2, asrc2, adst2, ms2, md2 = _tc_combine1(
        num1, den1, loops[0], loops[1], asrc1, adst1, xw1, ms1, md1, me1,
        b1lo, b1hi, wll, wlh, whl, whh, as2lo, as2hi, ad2lo, ad2hi)
    num2, den2 = _make_sc_edge_pass(False)(src, dst, asrc2, adst2, xw2,
                                           ms2, md2)
    h2, lp = _tc_combine2(num2, den2, asrc2, adst2, xw2, ms2, md2,
                          bias2.reshape(1, C))
    return (h2, lp)


# speedup vs baseline: 36.7024x; 1.1603x over previous
import functools

import jax
import jax.numpy as jnp
from jax import lax
from jax.experimental import pallas as pl
from jax.experimental.pallas import tpu as pltpu
from jax.experimental.pallas import tpu_sc as plsc

N = 10000
E = 320000
NFEAT = 128
H = 12
C = 16
HC = H * C
HH = H // 2
HW = HH * C

BN = 400
GN = N // BN
BE = 3200
GE = E // BE

NSC = 2
NTILE = 16
NW = NSC * NTILE
CH = 80
EPT_A = E // NW
NCH_A = EPT_A // CH
EPT_B = E // NTILE
NCH_B = EPT_B // CH
RPT = N // NTILE
WB = 125
NWB = RPT // WB

_f32 = jnp.float32

_SC_PARAMS = pltpu.CompilerParams(
    use_tc_tiling_on_sc=False, needs_layout_passes=False)


def _bd(rows, cols, shift):
    r = lax.broadcasted_iota(jnp.int32, (rows, cols), 0)
    c = lax.broadcasted_iota(jnp.int32, (rows, cols), 1)
    return (r // C + shift == c).astype(_f32)


def _bd_t(rows, cols, shift):
    r = lax.broadcasted_iota(jnp.int32, (rows, cols), 0)
    c = lax.broadcasted_iota(jnp.int32, (rows, cols), 1)
    return (c // C + shift == r).astype(_f32)


def _node_prologue_body(x_ref, wlo_ref, whi_ref, aslo_ref, ashi_ref,
                        adlo_ref, adhi_ref,
                        xw_ref, as_ref, ad_ref, ms_ref, md_ref):
    i = pl.program_id(0)
    xb = x_ref[...]
    xwlo = jnp.dot(xb, wlo_ref[...], preferred_element_type=_f32)
    xwhi = jnp.dot(xb, whi_ref[...], preferred_element_type=_f32)
    xw_ref[0] = xwlo
    xw_ref[1] = xwhi
    Blo = _bd(HW, C, 0)
    Bhi = _bd(HW, C, HH)
    a_s = (jnp.dot(xwlo * aslo_ref[...], Blo, preferred_element_type=_f32)
           + jnp.dot(xwhi * ashi_ref[...], Bhi, preferred_element_type=_f32))
    a_d = (jnp.dot(xwlo * adlo_ref[...], Blo, preferred_element_type=_f32)
           + jnp.dot(xwhi * adhi_ref[...], Bhi, preferred_element_type=_f32))
    as_ref[...] = a_s
    ad_ref[...] = a_d
    bs = jnp.broadcast_to(jnp.max(a_s, axis=0, keepdims=True), (8, C))
    bd = jnp.broadcast_to(jnp.max(a_d, axis=0, keepdims=True), (8, C))

    @pl.when(i == 0)
    def _():
        ms_ref[...] = bs
        md_ref[...] = bd

    @pl.when(i > 0)
    def _():
        ms_ref[...] = jnp.maximum(ms_ref[...], bs)
        md_ref[...] = jnp.maximum(md_ref[...], bd)


def _tc_node_prologue(x, wlo, whi, aslo, ashi, adlo, adhi):
    w_spec = pl.BlockSpec((NFEAT, HW), lambda i: (0, 0))
    v_spec = pl.BlockSpec((1, HW), lambda i: (0, 0))
    return pl.pallas_call(
        _node_prologue_body,
        grid=(GN,),
        in_specs=[
            pl.BlockSpec((BN, NFEAT), lambda i: (i, 0)),
            w_spec, w_spec, v_spec, v_spec, v_spec, v_spec,
        ],
        out_specs=[
            pl.BlockSpec((2, BN, HW), lambda i: (0, i, 0)),
            pl.BlockSpec((BN, C), lambda i: (i, 0)),
            pl.BlockSpec((BN, C), lambda i: (i, 0)),
            pl.BlockSpec((8, C), lambda i: (0, 0)),
            pl.BlockSpec((8, C), lambda i: (0, 0)),
        ],
        out_shape=[
            jax.ShapeDtypeStruct((2, N, HW), _f32),
            jax.ShapeDtypeStruct((N, C), _f32),
            jax.ShapeDtypeStruct((N, C), _f32),
            jax.ShapeDtypeStruct((8, C), _f32),
            jax.ShapeDtypeStruct((8, C), _f32),
        ],
    )(x, wlo, whi, aslo, ashi, adlo, adhi)


def _edge_prologue_body(ea_ref, we_ref, atte_ref, ae_ref, me_ref):
    i = pl.program_id(0)
    ew = jnp.dot(ea_ref[...], we_ref[...], preferred_element_type=_f32)
    B = _bd(HC, C, 0)
    ae = jnp.dot(ew * atte_ref[...], B, preferred_element_type=_f32)
    col = lax.broadcasted_iota(jnp.int32, (BE, C), 1)
    ae = ae + jnp.where(col == H, 1.0, 0.0).astype(_f32)
    ae_ref[...] = ae
    bm = jnp.broadcast_to(jnp.max(ae, axis=0, keepdims=True), (8, C))

    @pl.when(i == 0)
    def _():
        me_ref[...] = bm

    @pl.when(i > 0)
    def _():
        me_ref[...] = jnp.maximum(me_ref[...], bm)


def _tc_edge_prologue(ea, we, attef):
    return pl.pallas_call(
        _edge_prologue_body,
        grid=(GE,),
        in_specs=[
            pl.BlockSpec((BE, C), lambda i: (i, 0)),
            pl.BlockSpec((C, HC), lambda i: (0, 0)),
            pl.BlockSpec((1, HC), lambda i: (0, 0)),
        ],
        out_specs=[
            pl.BlockSpec((BE, C), lambda i: (i, 0)),
            pl.BlockSpec((8, C), lambda i: (0, 0)),
        ],
        out_shape=[
            jax.ShapeDtypeStruct((E, C), _f32),
            jax.ShapeDtypeStruct((8, C), _f32),
        ],
    )(ea, we, attef)


def _sc_loopsum_body(rows_hbm, idx_hbm, out_hbm, idxb, rowsb, wbuf, acc):
    c = lax.axis_index("c")
    s = lax.axis_index("s")
    tid = s * NSC + c

    def zrow(i, _):
        wbuf[i, :] = jnp.zeros((C,), _f32)
        return 0

    lax.fori_loop(0, WB, zrow, 0)
    for k in range(NWB):
        pltpu.sync_copy(wbuf, acc.at[pl.ds(s * RPT + k * WB, WB)])
    plsc.subcore_barrier()

    def chunk(j, _):
        base = tid * EPT_A + j * CH
        pltpu.sync_copy(idx_hbm.at[pl.ds(base, CH)], idxb)
        pltpu.sync_copy(rows_hbm.at[pl.ds(base, CH)], rowsb)
        pltpu.sync_copy(rowsb, acc.at[idxb], add=True)
        return 0

    lax.fori_loop(0, NCH_A, chunk, 0)
    plsc.subcore_barrier()
    for k in range(NWB):
        sl = pl.ds(s * RPT + k * WB, WB)
        pltpu.sync_copy(acc.at[sl], wbuf)
        pltpu.sync_copy(wbuf, out_hbm.at[c, sl])


@functools.cache
def _get_sc_loopsum():
    return pl.kernel(
        _sc_loopsum_body,
        out_type=jax.ShapeDtypeStruct((NSC, N, C), _f32),
        mesh=plsc.VectorSubcoreMesh(core_axis_name="c", subcore_axis_name="s",
                                    num_cores=NSC, num_subcores=NTILE),
        scratch_types=[
            pltpu.VMEM((CH,), jnp.int32),
            pltpu.VMEM((CH, C), _f32),
            pltpu.VMEM((WB, C), _f32),
            pltpu.VMEM_SHARED((N, C), _f32),
        ],
        compiler_params=_SC_PARAMS,
    )


def _sc_edge_pass_body(has_edge, *refs):
    if has_edge:
        (src_hbm, dst_hbm, as_hbm, ad_hbm, ae_hbm, xw_hbm, ms_hbm, md_hbm,
         me_hbm, num_out, den_out,
         idxs0, idxs1, idxd0, idxd1, idxc0, idxc1,
         rs0, rs1, rd0, rd1, re0, re1, xwr0, xwr1, outr0, outr1, exr0, exr1,
         m1, m2, m3, wbuf, wbden, accn, accd,
         gsem0, gsem1, ssem0, ssem1, isem0, isem1) = refs
        re_ = (re0, re1)
    else:
        (src_hbm, dst_hbm, as_hbm, ad_hbm, xw_hbm, ms_hbm, md_hbm,
         num_out, den_out,
         idxs0, idxs1, idxd0, idxd1, idxc0, idxc1,
         rs0, rs1, rd0, rd1, xwr0, xwr1, outr0, outr1, exr0, exr1,
         m1, m2, m3, wbuf, wbden, accn, accd,
         gsem0, gsem1, ssem0, ssem1, isem0, isem1) = refs
        re_ = (None, None)
    idxs_ = (idxs0, idxs1)
    idxd_ = (idxd0, idxd1)
    idxc_ = (idxc0, idxc1)
    rs_ = (rs0, rs1)
    rd_ = (rd0, rd1)
    xwr_ = (xwr0, xwr1)
    outr_ = (outr0, outr1)
    exr_ = (exr0, exr1)
    gsem_ = (gsem0, gsem1)
    ssem_ = (ssem0, ssem1)
    isem_ = (isem0, isem1)
    c = lax.axis_index("c")
    s = lax.axis_index("s")

    pltpu.sync_copy(ms_hbm, m1)
    pltpu.sync_copy(md_hbm, m2)
    S = m1[0, :] + m2[0, :]
    if has_edge:
        pltpu.sync_copy(me_hbm, m3)
        S = S + m3[0, :]
    S = jnp.maximum(S, 0.0)
    idxh_ = [jnp.full((C,), h, jnp.int32) + c * HH for h in range(HH)]

    def zrow(i, _):
        for j in range(HH):
            wbuf[i, pl.ds(j * C, C)] = jnp.zeros((C,), _f32)
        wbden[i, :] = jnp.zeros((C,), _f32)
        return 0

    lax.fori_loop(0, WB, zrow, 0)
    for k in range(NWB):
        sl = pl.ds(s * RPT + k * WB, WB)
        pltpu.sync_copy(wbuf, accn.at[sl])
        pltpu.sync_copy(wbden, accd.at[sl])
    plsc.subcore_barrier()

    def start_idx(j, b):
        base = s * EPT_B + j * CH
        pltpu.async_copy(src_hbm.at[pl.ds(base, CH)], idxs_[b], isem_[b])
        pltpu.async_copy(dst_hbm.at[pl.ds(base, CH)], idxd_[b], isem_[b])

    def wait_idx(b):
        pltpu.make_async_copy(
            src_hbm.at[pl.ds(0, CH)], idxs_[b], isem_[b]).wait()
        pltpu.make_async_copy(
            dst_hbm.at[pl.ds(0, CH)], idxd_[b], isem_[b]).wait()

    def start_gathers(j, b):
        base = s * EPT_B + j * CH
        pltpu.async_copy(as_hbm.at[idxs_[b]], rs_[b], gsem_[b])
        pltpu.async_copy(ad_hbm.at[idxd_[b]], rd_[b], gsem_[b])
        pltpu.async_copy(xw_hbm.at[c].at[idxs_[b]], xwr_[b], gsem_[b])
        if has_edge:
            pltpu.async_copy(ae_hbm.at[pl.ds(base, CH)], re_[b], gsem_[b])

    def wait_gathers(b):
        pltpu.make_async_copy(as_hbm.at[idxs_[b]], rs_[b], gsem_[b]).wait()
        pltpu.make_async_copy(ad_hbm.at[idxd_[b]], rd_[b], gsem_[b]).wait()
        pltpu.make_async_copy(
            xw_hbm.at[c].at[idxs_[b]], xwr_[b], gsem_[b]).wait()
        if has_edge:
            pltpu.make_async_copy(
                ae_hbm.at[pl.ds(0, CH)], re_[b], gsem_[b]).wait()

    def wait_scatters(b):
        pltpu.make_async_copy(
            outr_[b], accn.at[idxc_[b]], ssem_[b]).wait()

        @pl.when(c == 0)
        def _():
            pltpu.make_async_copy(
                exr_[b], accd.at[idxc_[b]], ssem_[b]).wait()

    start_idx(0, 0)
    start_idx(1, 1)
    wait_idx(0)
    start_gathers(0, 0)

    def outer(jo, _):
        for b in (0, 1):
            j = 2 * jo + b
            @pl.when(j + 1 < NCH_B)
            def _():
                wait_idx(1 - b)
                start_gathers(j + 1, 1 - b)

            wait_gathers(b)

            @pl.when(jo > 0)
            def _():
                wait_scatters(b)

            for k in range(CH // C):
                idxc_[b][pl.ds(k * C, C)] = idxd_[b][pl.ds(k * C, C)]

            @pl.when(j + 2 < NCH_B)
            def _():
                start_idx(j + 2, b)

            rs, rd, re = rs_[b], rd_[b], re_[b]
            xwr, outr, exr = xwr_[b], outr_[b], exr_[b]

            def edge_ex(e, _):
                a = rs[e, :] + rd[e, :]
                if has_edge:
                    a = a + re[e, :]
                a = jnp.where(a >= 0.0, a, 0.2 * a)
                exr[e, :] = jnp.exp(a - S)
                return 0

            lax.fori_loop(0, CH, edge_ex, 0, unroll=4)

            def edge_mul(e, _):
                idxe = jnp.full((C,), e, jnp.int32)
                for h in range(HH):
                    bc = plsc.load_gather(exr, [idxe, idxh_[h]])
                    outr[e, pl.ds(h * C, C)] = xwr[e, pl.ds(h * C, C)] * bc
                return 0

            lax.fori_loop(0, CH, edge_mul, 0, unroll=4)
            pltpu.async_copy(outr, accn.at[idxc_[b]], ssem_[b], add=True)

            @pl.when(c == 0)
            def _():
                pltpu.async_copy(exr, accd.at[idxc_[b]], ssem_[b], add=True)

        return 0

    lax.fori_loop(0, NCH_B // 2, outer, 0)
    wait_scatters(0)
    wait_scatters(1)
    plsc.subcore_barrier()
    for k in range(NWB):
        sl = pl.ds(s * RPT + k * WB, WB)
        pltpu.sync_copy(accn.at[sl], wbuf)
        pltpu.sync_copy(wbuf, num_out.at[c, sl])

    @pl.when(c == 0)
    def _():
        for k in range(NWB):
            sl = pl.ds(s * RPT + k * WB, WB)
            pltpu.sync_copy(accd.at[sl], wbden)
            pltpu.sync_copy(wbden, den_out.at[sl])


@functools.cache
def _make_sc_edge_pass(has_edge):
    scratch = [
        pltpu.VMEM((CH,), jnp.int32),
        pltpu.VMEM((CH,), jnp.int32),
        pltpu.VMEM((CH,), jnp.int32),
        pltpu.VMEM((CH,), jnp.int32),
        pltpu.VMEM((CH,), jnp.int32),
        pltpu.VMEM((CH,), jnp.int32),
        pltpu.VMEM((CH, C), _f32),
        pltpu.VMEM((CH, C), _f32),
        pltpu.VMEM((CH, C), _f32),
        pltpu.VMEM((CH, C), _f32),
    ]
    if has_edge:
        scratch += [pltpu.VMEM((CH, C), _f32),
                    pltpu.VMEM((CH, C), _f32)]
    scratch += [
        pltpu.VMEM((CH, HW), _f32),
        pltpu.VMEM((CH, HW), _f32),
        pltpu.VMEM((CH, HW), _f32),
        pltpu.VMEM((CH, HW), _f32),
        pltpu.VMEM((CH, C), _f32),
        pltpu.VMEM((CH, C), _f32),
        pltpu.VMEM((8, C), _f32),
        pltpu.VMEM((8, C), _f32),
        pltpu.VMEM((8, C), _f32),
        pltpu.VMEM((WB, HW), _f32),
        pltpu.VMEM((WB, C), _f32),
        pltpu.VMEM_SHARED((N, HW), _f32),
        pltpu.VMEM_SHARED((N, C), _f32),
        pltpu.SemaphoreType.DMA,
        pltpu.SemaphoreType.DMA,
        pltpu.SemaphoreType.DMA,
        pltpu.SemaphoreType.DMA,
        pltpu.SemaphoreType.DMA,
        pltpu.SemaphoreType.DMA,
    ]
    return pl.kernel(
        functools.partial(_sc_edge_pass_body, has_edge),
        out_type=(
            jax.ShapeDtypeStruct((NSC, N, HW), _f32),
            jax.ShapeDtypeStruct((N, C), _f32),
        ),
        mesh=plsc.VectorSubcoreMesh(core_axis_name="c", subcore_axis_name="s",
                                    num_cores=NSC, num_subcores=NTILE),
        scratch_types=scratch,
        compiler_params=_SC_PARAMS,
    )


def _combine1_body(nm_ref, dn_ref, l0_ref, l1_ref,
                   as_ref, ad_ref, xw_ref, ms_ref, md_ref, me_ref,
                   b1lo_ref, b1hi_ref, wll_ref, wlh_ref, whl_ref, whh_ref,
                   as2lo_ref, as2hi_ref, ad2lo_ref, ad2hi_ref,
                   xw2_ref, as2_ref, ad2_ref, ms2_ref, md2_ref):
    i = pl.program_id(0)
    den = dn_ref[...]
    ls = l0_ref[...] + l1_ref[...]
    onehot12 = jnp.where(
        lax.broadcasted_iota(jnp.int32, (1, C), 1) == H, 1.0, 0.0).astype(_f32)
    cnt = jnp.sum(ls * onehot12, axis=1, keepdims=True)
    lae = ls / jnp.maximum(cnt, 1.0)
    S = jnp.maximum(ms_ref[0:1, :] + md_ref[0:1, :] + me_ref[0:1, :], 0.0)
    al = as_ref[...] + ad_ref[...] + lae
    al = jnp.where(al >= 0.0, al, 0.2 * al)
    exl = jnp.exp(al - S)
    Rlo = _bd_t(C, HW, 0)
    Rhi = _bd_t(C, HW, HH)
    den_t = den + exl
    dinv = 1.0 / (den_t + 1e-16)
    hs = []
    for half, R, b1 in ((0, Rlo, b1lo_ref), (1, Rhi, b1hi_ref)):
        exb = jnp.dot(exl, R, preferred_element_type=_f32)
        num_t = nm_ref[half] + xw_ref[half] * exb
        dinvb = jnp.dot(dinv, R, preferred_element_type=_f32)
        hv = num_t * dinvb + b1[...]
        hs.append(jnp.where(hv > 0.0, hv, jnp.exp(hv) - 1.0))
    xw2lo = (jnp.dot(hs[0], wll_ref[...], preferred_element_type=_f32)
             + jnp.dot(hs[1], whl_ref[...], preferred_element_type=_f32))
    xw2hi = (jnp.dot(hs[0], wlh_ref[...], preferred_element_type=_f32)
             + jnp.dot(hs[1], whh_ref[...], preferred_element_type=_f32))
    xw2_ref[0] = xw2lo
    xw2_ref[1] = xw2hi
    Blo = _bd(HW, C, 0)
    Bhi = _bd(HW, C, HH)
    a_s2 = (jnp.dot(xw2lo * as2lo_ref[...], Blo, preferred_element_type=_f32)
            + jnp.dot(xw2hi * as2hi_ref[...], Bhi,
                      preferred_element_type=_f32))
    a_d2 = (jnp.dot(xw2lo * ad2lo_ref[...], Blo, preferred_element_type=_f32)
            + jnp.dot(xw2hi * ad2hi_ref[...], Bhi,
                      preferred_element_type=_f32))
    as2_ref[...] = a_s2
    ad2_ref[...] = a_d2
    bs = jnp.broadcast_to(jnp.max(a_s2, axis=0, keepdims=True), (8, C))
    bd = jnp.broadcast_to(jnp.max(a_d2, axis=0, keepdims=True), (8, C))

    @pl.when(i == 0)
    def _():
        ms2_ref[...] = bs
        md2_ref[...] = bd

    @pl.when(i > 0)
    def _():
        ms2_ref[...] = jnp.maximum(ms2_ref[...], bs)
        md2_ref[...] = jnp.maximum(md2_ref[...], bd)


def _tc_combine1(nm, dn, l0, l1, asrc, adst, xw, ms, md, me,
                 b1lo, b1hi, wll, wlh, whl, whh,
                 as2lo, as2hi, ad2lo, ad2hi):
    spec_cat = pl.BlockSpec((2, BN, HW), lambda i: (0, i, 0))
    spec_n16 = pl.BlockSpec((BN, C), lambda i: (i, 0))
    spec_m = pl.BlockSpec((8, C), lambda i: (0, 0))
    spec_v = pl.BlockSpec((1, HW), lambda i: (0, 0))
    spec_w = pl.BlockSpec((HW, HW), lambda i: (0, 0))
    return pl.pallas_call(
        _combine1_body,
        grid=(GN,),
        in_specs=[
            spec_cat, spec_n16, spec_n16, spec_n16,
            spec_n16, spec_n16, spec_cat,
            spec_m, spec_m, spec_m,
            spec_v, spec_v, spec_w, spec_w, spec_w, spec_w,
            spec_v, spec_v, spec_v, spec_v,
        ],
        out_specs=[spec_cat, spec_n16, spec_n16, spec_m, spec_m],
        out_shape=[
            jax.ShapeDtypeStruct((2, N, HW), _f32),
            jax.ShapeDtypeStruct((N, C), _f32),
            jax.ShapeDtypeStruct((N, C), _f32),
            jax.ShapeDtypeStruct((8, C), _f32),
            jax.ShapeDtypeStruct((8, C), _f32),
        ],
    )(nm, dn, l0, l1, asrc, adst, xw, ms, md, me,
      b1lo, b1hi, wll, wlh, whl, whh, as2lo, as2hi, ad2lo, ad2hi)


def _combine2_body(nm_ref, dn_ref, as_ref, ad_ref, xw_ref,
                   ms_ref, md_ref, b2_ref, h2_ref, lp_ref):
    den = dn_ref[...]
    S = jnp.maximum(ms_ref[0:1, :] + md_ref[0:1, :], 0.0)
    al = as_ref[...] + ad_ref[...]
    al = jnp.where(al >= 0.0, al, 0.2 * al)
    exl = jnp.exp(al - S)
    den_t = den + exl
    dinv = 1.0 / (den_t + 1e-16)
    rr = lax.broadcasted_iota(jnp.int32, (HW, C), 0)
    cc = lax.broadcasted_iota(jnp.int32, (HW, C), 1)
    Rm = jnp.where(rr % C == cc, 1.0 / H, 0.0).astype(_f32)
    h2 = b2_ref[...]
    for half, R in ((0, _bd_t(C, HW, 0)), (1, _bd_t(C, HW, HH))):
        exb = jnp.dot(exl, R, preferred_element_type=_f32)
        num_t = nm_ref[half] + xw_ref[half] * exb
        dinvb = jnp.dot(dinv, R, preferred_element_type=_f32)
        h2 = h2 + jnp.dot(num_t * dinvb, Rm, preferred_element_type=_f32)
    h2_ref[...] = h2
    m = jnp.max(h2, axis=1, keepdims=True)
    z = h2 - m
    lse = jnp.log(jnp.sum(jnp.exp(z), axis=1, keepdims=True))
    lp_ref[...] = z - lse


def _tc_combine2(nm, dn, asrc, adst, xw, ms, md, b2r):
    spec_cat = pl.BlockSpec((2, BN, HW), lambda i: (0, i, 0))
    spec_n16 = pl.BlockSpec((BN, C), lambda i: (i, 0))
    spec_m = pl.BlockSpec((8, C), lambda i: (0, 0))
    return pl.pallas_call(
        _combine2_body,
        grid=(GN,),
        in_specs=[
            spec_cat, spec_n16, spec_n16, spec_n16, spec_cat,
            spec_m, spec_m,
            pl.BlockSpec((1, C), lambda i: (0, 0)),
        ],
        out_specs=[spec_n16, spec_n16],
        out_shape=[
            jax.ShapeDtypeStruct((N, C), _f32),
            jax.ShapeDtypeStruct((N, C), _f32),
        ],
    )(nm, dn, asrc, adst, xw, ms, md, b2r)


def kernel(x, edge_index, edge_attr, W1, att_src1, att_dst1, W_edge1,
           att_edge1, bias1, W2, att_src2, att_dst2, bias2):
    src = edge_index[0]
    dst = edge_index[1]
    w1lo, w1hi = W1[:, :HW], W1[:, HW:]
    as1lo = att_src1[:HH].reshape(1, HW)
    as1hi = att_src1[HH:].reshape(1, HW)
    ad1lo = att_dst1[:HH].reshape(1, HW)
    ad1hi = att_dst1[HH:].reshape(1, HW)
    attef1 = att_edge1.reshape(1, HC)
    as2lo = att_src2[:HH].reshape(1, HW)
    as2hi = att_src2[HH:].reshape(1, HW)
    ad2lo = att_dst2[:HH].reshape(1, HW)
    ad2hi = att_dst2[HH:].reshape(1, HW)
    wll, wlh = W2[:HW, :HW], W2[:HW, HW:]
    whl, whh = W2[HW:, :HW], W2[HW:, HW:]
    b1lo = bias1[:HW].reshape(1, HW)
    b1hi = bias1[HW:].reshape(1, HW)

    xw1, asrc1, adst1, ms1, md1 = _tc_node_prologue(
        x, w1lo, w1hi, as1lo, as1hi, ad1lo, ad1hi)
    ae1, me1 = _tc_edge_prologue(edge_attr, W_edge1, attef1)
    loops = _get_sc_loopsum()(ae1, dst)
    num1, den1 = _make_sc_edge_pass(True)(src, dst, asrc1, adst1, ae1, xw1,
                                          ms1, md1, me1)
    xw2, asrc2, adst2, ms2, md2 = _tc_combine1(
        num1, den1, loops[0], loops[1], asrc1, adst1, xw1, ms1, md1, me1,
        b1lo, b1hi, wll, wlh, whl, whh, as2lo, as2hi, ad2lo, ad2hi)
    num2, den2 = _make_sc_edge_pass(False)(src, dst, asrc2, adst2, xw2,
                                           ms2, md2)
    h2, lp = _tc_combine2(num2, den2, asrc2, adst2, xw2, ms2, md2,
                          bias2.reshape(1, C))
    return (h2, lp)


# speedup vs baseline: 39.4117x; 1.0738x over previous
import functools

import jax
import jax.numpy as jnp
from jax import lax
from jax.experimental import pallas as pl
from jax.experimental.pallas import tpu as pltpu
from jax.experimental.pallas import tpu_sc as plsc

N = 10000
E = 320000
NFEAT = 128
H = 12
C = 16
HC = H * C
HH = H // 2
HW = HH * C

BN = 400
GN = N // BN
BE = 8000
GE = E // BE

NSC = 2
NTILE = 16
NW = NSC * NTILE
CH = 80
EPT_A = E // NW
NCH_A = EPT_A // CH
EPT_B = E // NTILE
NCH_B = EPT_B // CH
RPT = N // NTILE
WB = 125
NWB = RPT // WB

_f32 = jnp.float32

_SC_PARAMS = pltpu.CompilerParams(
    use_tc_tiling_on_sc=False, needs_layout_passes=False)


def _bd(rows, cols, shift):
    r = lax.broadcasted_iota(jnp.int32, (rows, cols), 0)
    c = lax.broadcasted_iota(jnp.int32, (rows, cols), 1)
    return (r // C + shift == c).astype(_f32)


def _bd_t(rows, cols, shift):
    r = lax.broadcasted_iota(jnp.int32, (rows, cols), 0)
    c = lax.broadcasted_iota(jnp.int32, (rows, cols), 1)
    return (c // C + shift == r).astype(_f32)


def _node_prologue_body(x_ref, wlo_ref, whi_ref, aslo_ref, ashi_ref,
                        adlo_ref, adhi_ref,
                        xw_ref, as_ref, ad_ref, ms_ref, md_ref):
    i = pl.program_id(0)
    xb = x_ref[...]
    xwlo = jnp.dot(xb, wlo_ref[...], preferred_element_type=_f32)
    xwhi = jnp.dot(xb, whi_ref[...], preferred_element_type=_f32)
    xw_ref[0] = xwlo
    xw_ref[1] = xwhi
    Blo = _bd(HW, C, 0)
    Bhi = _bd(HW, C, HH)
    a_s = (jnp.dot(xwlo * aslo_ref[...], Blo, preferred_element_type=_f32)
           + jnp.dot(xwhi * ashi_ref[...], Bhi, preferred_element_type=_f32))
    a_d = (jnp.dot(xwlo * adlo_ref[...], Blo, preferred_element_type=_f32)
           + jnp.dot(xwhi * adhi_ref[...], Bhi, preferred_element_type=_f32))
    as_ref[...] = a_s
    ad_ref[...] = a_d
    bs = jnp.broadcast_to(jnp.max(a_s, axis=0, keepdims=True), (8, C))
    bd = jnp.broadcast_to(jnp.max(a_d, axis=0, keepdims=True), (8, C))

    @pl.when(i == 0)
    def _():
        ms_ref[...] = bs
        md_ref[...] = bd

    @pl.when(i > 0)
    def _():
        ms_ref[...] = jnp.maximum(ms_ref[...], bs)
        md_ref[...] = jnp.maximum(md_ref[...], bd)


def _tc_node_prologue(x, wlo, whi, aslo, ashi, adlo, adhi):
    w_spec = pl.BlockSpec((NFEAT, HW), lambda i: (0, 0))
    v_spec = pl.BlockSpec((1, HW), lambda i: (0, 0))
    return pl.pallas_call(
        _node_prologue_body,
        grid=(GN,),
        in_specs=[
            pl.BlockSpec((BN, NFEAT), lambda i: (i, 0)),
            w_spec, w_spec, v_spec, v_spec, v_spec, v_spec,
        ],
        out_specs=[
            pl.BlockSpec((2, BN, HW), lambda i: (0, i, 0)),
            pl.BlockSpec((BN, C), lambda i: (i, 0)),
            pl.BlockSpec((BN, C), lambda i: (i, 0)),
            pl.BlockSpec((8, C), lambda i: (0, 0)),
            pl.BlockSpec((8, C), lambda i: (0, 0)),
        ],
        out_shape=[
            jax.ShapeDtypeStruct((2, N, HW), _f32),
            jax.ShapeDtypeStruct((N, C), _f32),
            jax.ShapeDtypeStruct((N, C), _f32),
            jax.ShapeDtypeStruct((8, C), _f32),
            jax.ShapeDtypeStruct((8, C), _f32),
        ],
    )(x, wlo, whi, aslo, ashi, adlo, adhi)


def _edge_prologue_body(ea_ref, we_ref, atte_ref, ae_ref, me_ref):
    i = pl.program_id(0)
    B = _bd(HC, C, 0)
    Me = jnp.dot(we_ref[...] * atte_ref[...], B, preferred_element_type=_f32)
    ae = jnp.dot(ea_ref[...], Me, preferred_element_type=_f32)
    col = lax.broadcasted_iota(jnp.int32, (BE, C), 1)
    ae = ae + jnp.where(col == H, 1.0, 0.0).astype(_f32)
    ae_ref[...] = ae
    bm = jnp.broadcast_to(jnp.max(ae, axis=0, keepdims=True), (8, C))

    @pl.when(i == 0)
    def _():
        me_ref[...] = bm

    @pl.when(i > 0)
    def _():
        me_ref[...] = jnp.maximum(me_ref[...], bm)


def _tc_edge_prologue(ea, we, attef):
    return pl.pallas_call(
        _edge_prologue_body,
        grid=(GE,),
        in_specs=[
            pl.BlockSpec((BE, C), lambda i: (i, 0)),
            pl.BlockSpec((C, HC), lambda i: (0, 0)),
            pl.BlockSpec((1, HC), lambda i: (0, 0)),
        ],
        out_specs=[
            pl.BlockSpec((BE, C), lambda i: (i, 0)),
            pl.BlockSpec((8, C), lambda i: (0, 0)),
        ],
        out_shape=[
            jax.ShapeDtypeStruct((E, C), _f32),
            jax.ShapeDtypeStruct((8, C), _f32),
        ],
    )(ea, we, attef)


def _sc_edge_pass_body(has_edge, *refs):
    if has_edge:
        (src_hbm, dst_hbm, as_hbm, ad_hbm, ae_hbm, xw_hbm, ms_hbm, md_hbm,
         me_hbm, num_out, den_out, loop_out,
         idxs0, idxs1, idxd0, idxd1, idxc0, idxc1,
         rs0, rs1, rd0, rd1, re0, re1, rec0, rec1,
         xwr0, xwr1, outr0, outr1, exr0, exr1,
         m1, m2, m3, accn, accd, accl,
         gsem0, gsem1, ssem0, ssem1, isem0, isem1) = refs
        re_ = (re0, re1)
        rec_ = (rec0, rec1)
    else:
        (src_hbm, dst_hbm, as_hbm, ad_hbm, xw_hbm, ms_hbm, md_hbm,
         num_out, den_out,
         idxs0, idxs1, idxd0, idxd1, idxc0, idxc1,
         rs0, rs1, rd0, rd1, xwr0, xwr1, outr0, outr1, exr0, exr1,
         m1, m2, m3, accn, accd,
         gsem0, gsem1, ssem0, ssem1, isem0, isem1) = refs
        re_ = (None, None)
        rec_ = (None, None)
        accl = None
        loop_out = None
    idxs_ = (idxs0, idxs1)
    idxd_ = (idxd0, idxd1)
    idxc_ = (idxc0, idxc1)
    rs_ = (rs0, rs1)
    rd_ = (rd0, rd1)
    xwr_ = (xwr0, xwr1)
    outr_ = (outr0, outr1)
    exr_ = (exr0, exr1)
    gsem_ = (gsem0, gsem1)
    ssem_ = (ssem0, ssem1)
    isem_ = (isem0, isem1)
    c = lax.axis_index("c")
    s = lax.axis_index("s")

    pltpu.sync_copy(ms_hbm, m1)
    pltpu.sync_copy(md_hbm, m2)
    S = m1[0, :] + m2[0, :]
    if has_edge:
        pltpu.sync_copy(me_hbm, m3)
        S = S + m3[0, :]
    S = jnp.maximum(S, 0.0)
    idxh_ = [jnp.full((C,), h, jnp.int32) + c * HH for h in range(HH)]

    def zrow(i, _):
        for jj in range(HH):
            xwr0[i, pl.ds(jj * C, C)] = jnp.zeros((C,), _f32)
        rs0[i, :] = jnp.zeros((C,), _f32)
        return 0

    lax.fori_loop(0, CH, zrow, 0)
    segs = [(k * CH, CH) for k in range(RPT // CH)] + [
        (RPT - RPT % CH, RPT % CH)]
    for off, sz in segs:
        sl = pl.ds(s * RPT + off, sz)
        pltpu.sync_copy(xwr0.at[pl.ds(0, sz)], accn.at[sl])
        pltpu.sync_copy(rs0.at[pl.ds(0, sz)], accd.at[sl])
        if has_edge:
            pltpu.sync_copy(rs0.at[pl.ds(0, sz)], accl.at[sl])
    plsc.subcore_barrier()

    def start_idx(j, b):
        base = s * EPT_B + j * CH
        pltpu.async_copy(src_hbm.at[pl.ds(base, CH)], idxs_[b], isem_[b])
        pltpu.async_copy(dst_hbm.at[pl.ds(base, CH)], idxd_[b], isem_[b])

    def wait_idx(b):
        pltpu.make_async_copy(
            src_hbm.at[pl.ds(0, CH)], idxs_[b], isem_[b]).wait()
        pltpu.make_async_copy(
            dst_hbm.at[pl.ds(0, CH)], idxd_[b], isem_[b]).wait()

    def start_gathers(j, b):
        base = s * EPT_B + j * CH
        pltpu.async_copy(as_hbm.at[idxs_[b]], rs_[b], gsem_[b])
        pltpu.async_copy(ad_hbm.at[idxd_[b]], rd_[b], gsem_[b])
        pltpu.async_copy(xw_hbm.at[c].at[idxs_[b]], xwr_[b], gsem_[b])
        if has_edge:
            pltpu.async_copy(ae_hbm.at[pl.ds(base, CH)], re_[b], gsem_[b])

    def wait_gathers(b):
        pltpu.make_async_copy(as_hbm.at[idxs_[b]], rs_[b], gsem_[b]).wait()
        pltpu.make_async_copy(ad_hbm.at[idxd_[b]], rd_[b], gsem_[b]).wait()
        pltpu.make_async_copy(
            xw_hbm.at[c].at[idxs_[b]], xwr_[b], gsem_[b]).wait()
        if has_edge:
            pltpu.make_async_copy(
                ae_hbm.at[pl.ds(0, CH)], re_[b], gsem_[b]).wait()

    def wait_scatters(b):
        pltpu.make_async_copy(
            outr_[b], accn.at[idxc_[b]], ssem_[b]).wait()

        @pl.when(c == 0)
        def _():
            pltpu.make_async_copy(
                exr_[b], accd.at[idxc_[b]], ssem_[b]).wait()

        if has_edge:
            @pl.when(c == 1)
            def _():
                pltpu.make_async_copy(
                    rec_[b], accl.at[idxc_[b]], ssem_[b]).wait()

    start_idx(0, 0)
    start_idx(1, 1)
    wait_idx(0)
    start_gathers(0, 0)

    def outer(jo, _):
        for b in (0, 1):
            j = 2 * jo + b
            @pl.when(j + 1 < NCH_B)
            def _():
                wait_idx(1 - b)
                start_gathers(j + 1, 1 - b)

            wait_gathers(b)

            @pl.when(jo > 0)
            def _():
                wait_scatters(b)

            for k in range(CH // C):
                idxc_[b][pl.ds(k * C, C)] = idxd_[b][pl.ds(k * C, C)]

            @pl.when(j + 2 < NCH_B)
            def _():
                start_idx(j + 2, b)

            rs, rd, re = rs_[b], rd_[b], re_[b]
            xwr, outr, exr = xwr_[b], outr_[b], exr_[b]

            def edge_ex(e, _):
                a = rs[e, :] + rd[e, :]
                if has_edge:
                    a = a + re[e, :]
                a = jnp.where(a >= 0.0, a, 0.2 * a)
                exr[e, :] = jnp.exp(a - S)
                return 0

            lax.fori_loop(0, CH, edge_ex, 0, unroll=4)

            def edge_mul(e, _):
                idxe = jnp.full((C,), e, jnp.int32)
                for h in range(HH):
                    bc = plsc.load_gather(exr, [idxe, idxh_[h]])
                    outr[e, pl.ds(h * C, C)] = xwr[e, pl.ds(h * C, C)] * bc
                return 0

            lax.fori_loop(0, CH, edge_mul, 0, unroll=4)
            pltpu.async_copy(outr, accn.at[idxc_[b]], ssem_[b], add=True)

            @pl.when(c == 0)
            def _():
                pltpu.async_copy(exr, accd.at[idxc_[b]], ssem_[b], add=True)

            if has_edge:
                @pl.when(c == 1)
                def _():
                    def cprow(e, _):
                        rec_[b][e, :] = re[e, :]
                        return 0

                    lax.fori_loop(0, CH, cprow, 0, unroll=8)
                    pltpu.async_copy(
                        rec_[b], accl.at[idxc_[b]], ssem_[b], add=True)

        return 0

    lax.fori_loop(0, NCH_B // 2, outer, 0)
    wait_scatters(0)
    wait_scatters(1)
    plsc.subcore_barrier()
    for off, sz in segs:
        sl = pl.ds(s * RPT + off, sz)
        pltpu.sync_copy(accn.at[sl], xwr0.at[pl.ds(0, sz)])
        pltpu.sync_copy(xwr0.at[pl.ds(0, sz)], num_out.at[c, sl])

    @pl.when(c == 0)
    def _():
        for off, sz in segs:
            sl = pl.ds(s * RPT + off, sz)
            pltpu.sync_copy(accd.at[sl], rs0.at[pl.ds(0, sz)])
            pltpu.sync_copy(rs0.at[pl.ds(0, sz)], den_out.at[sl])

    if has_edge:
        @pl.when(c == 1)
        def _():
            for off, sz in segs:
                sl = pl.ds(s * RPT + off, sz)
                pltpu.sync_copy(accl.at[sl], rd0.at[pl.ds(0, sz)])
                pltpu.sync_copy(rd0.at[pl.ds(0, sz)], loop_out.at[sl])


@functools.cache
def _make_sc_edge_pass(has_edge):
    scratch = [
        pltpu.VMEM((CH,), jnp.int32),
        pltpu.VMEM((CH,), jnp.int32),
        pltpu.VMEM((CH,), jnp.int32),
        pltpu.VMEM((CH,), jnp.int32),
        pltpu.VMEM((CH,), jnp.int32),
        pltpu.VMEM((CH,), jnp.int32),
        pltpu.VMEM((CH, C), _f32),
        pltpu.VMEM((CH, C), _f32),
        pltpu.VMEM((CH, C), _f32),
        pltpu.VMEM((CH, C), _f32),
    ]
    if has_edge:
        scratch += [pltpu.VMEM((CH, C), _f32),
                    pltpu.VMEM((CH, C), _f32),
                    pltpu.VMEM((CH, C), _f32),
                    pltpu.VMEM((CH, C), _f32)]
    scratch += [
        pltpu.VMEM((CH, HW), _f32),
        pltpu.VMEM((CH, HW), _f32),
        pltpu.VMEM((CH, HW), _f32),
        pltpu.VMEM((CH, HW), _f32),
        pltpu.VMEM((CH, C), _f32),
        pltpu.VMEM((CH, C), _f32),
        pltpu.VMEM((8, C), _f32),
        pltpu.VMEM((8, C), _f32),
        pltpu.VMEM((8, C), _f32),
        pltpu.VMEM_SHARED((N, HW), _f32),
        pltpu.VMEM_SHARED((N, C), _f32),
    ]
    if has_edge:
        scratch.append(pltpu.VMEM_SHARED((N, C), _f32))
    scratch += [
        pltpu.SemaphoreType.DMA,
        pltpu.SemaphoreType.DMA,
        pltpu.SemaphoreType.DMA,
        pltpu.SemaphoreType.DMA,
        pltpu.SemaphoreType.DMA,
        pltpu.SemaphoreType.DMA,
    ]
    outs = [
        jax.ShapeDtypeStruct((NSC, N, HW), _f32),
        jax.ShapeDtypeStruct((N, C), _f32),
    ]
    if has_edge:
        outs.append(jax.ShapeDtypeStruct((N, C), _f32))
    return pl.kernel(
        functools.partial(_sc_edge_pass_body, has_edge),
        out_type=tuple(outs),
        mesh=plsc.VectorSubcoreMesh(core_axis_name="c", subcore_axis_name="s",
                                    num_cores=NSC, num_subcores=NTILE),
        scratch_types=scratch,
        compiler_params=_SC_PARAMS,
    )


def _combine1_body(nm_ref, dn_ref, lp_ref,
                   as_ref, ad_ref, xw_ref, ms_ref, md_ref, me_ref,
                   b1lo_ref, b1hi_ref, wll_ref, wlh_ref, whl_ref, whh_ref,
                   as2lo_ref, as2hi_ref, ad2lo_ref, ad2hi_ref,
                   xw2_ref, as2_ref, ad2_ref, ms2_ref, md2_ref):
    i = pl.program_id(0)
    den = dn_ref[...]
    ls = lp_ref[...]
    onehot12 = jnp.where(
        lax.broadcasted_iota(jnp.int32, (1, C), 1) == H, 1.0, 0.0).astype(_f32)
    cnt = jnp.sum(ls * onehot12, axis=1, keepdims=True)
    lae = ls / jnp.maximum(cnt, 1.0)
    S = jnp.maximum(ms_ref[0:1, :] + md_ref[0:1, :] + me_ref[0:1, :], 0.0)
    al = as_ref[...] + ad_ref[...] + lae
    al = jnp.where(al >= 0.0, al, 0.2 * al)
    exl = jnp.exp(al - S)
    Rlo = _bd_t(C, HW, 0)
    Rhi = _bd_t(C, HW, HH)
    den_t = den + exl
    dinv = 1.0 / (den_t + 1e-16)
    hs = []
    for half, R, b1 in ((0, Rlo, b1lo_ref), (1, Rhi, b1hi_ref)):
        exb = jnp.dot(exl, R, preferred_element_type=_f32)
        num_t = nm_ref[half] + xw_ref[half] * exb
        dinvb = jnp.dot(dinv, R, preferred_element_type=_f32)
        hv = num_t * dinvb + b1[...]
        hs.append(jnp.where(hv > 0.0, hv, jnp.exp(hv) - 1.0))
    xw2lo = (jnp.dot(hs[0], wll_ref[...], preferred_element_type=_f32)
             + jnp.dot(hs[1], whl_ref[...], preferred_element_type=_f32))
    xw2hi = (jnp.dot(hs[0], wlh_ref[...], preferred_element_type=_f32)
             + jnp.dot(hs[1], whh_ref[...], preferred_element_type=_f32))
    xw2_ref[0] = xw2lo
    xw2_ref[1] = xw2hi
    Blo = _bd(HW, C, 0)
    Bhi = _bd(HW, C, HH)
    a_s2 = (jnp.dot(xw2lo * as2lo_ref[...], Blo, preferred_element_type=_f32)
            + jnp.dot(xw2hi * as2hi_ref[...], Bhi,
                      preferred_element_type=_f32))
    a_d2 = (jnp.dot(xw2lo * ad2lo_ref[...], Blo, preferred_element_type=_f32)
            + jnp.dot(xw2hi * ad2hi_ref[...], Bhi,
                      preferred_element_type=_f32))
    as2_ref[...] = a_s2
    ad2_ref[...] = a_d2
    bs = jnp.broadcast_to(jnp.max(a_s2, axis=0, keepdims=True), (8, C))
    bd = jnp.broadcast_to(jnp.max(a_d2, axis=0, keepdims=True), (8, C))

    @pl.when(i == 0)
    def _():
        ms2_ref[...] = bs
        md2_ref[...] = bd

    @pl.when(i > 0)
    def _():
        ms2_ref[...] = jnp.maximum(ms2_ref[...], bs)
        md2_ref[...] = jnp.maximum(md2_ref[...], bd)


def _tc_combine1(nm, dn, lp, asrc, adst, xw, ms, md, me,
                 b1lo, b1hi, wll, wlh, whl, whh,
                 as2lo, as2hi, ad2lo, ad2hi):
    spec_cat = pl.BlockSpec((2, BN, HW), lambda i: (0, i, 0))
    spec_n16 = pl.BlockSpec((BN, C), lambda i: (i, 0))
    spec_m = pl.BlockSpec((8, C), lambda i: (0, 0))
    spec_v = pl.BlockSpec((1, HW), lambda i: (0, 0))
    spec_w = pl.BlockSpec((HW, HW), lambda i: (0, 0))
    return pl.pallas_call(
        _combine1_body,
        grid=(GN,),
        in_specs=[
            spec_cat, spec_n16, spec_n16,
            spec_n16, spec_n16, spec_cat,
            spec_m, spec_m, spec_m,
            spec_v, spec_v, spec_w, spec_w, spec_w, spec_w,
            spec_v, spec_v, spec_v, spec_v,
        ],
        out_specs=[spec_cat, spec_n16, spec_n16, spec_m, spec_m],
        out_shape=[
            jax.ShapeDtypeStruct((2, N, HW), _f32),
            jax.ShapeDtypeStruct((N, C), _f32),
            jax.ShapeDtypeStruct((N, C), _f32),
            jax.ShapeDtypeStruct((8, C), _f32),
            jax.ShapeDtypeStruct((8, C), _f32),
        ],
    )(nm, dn, lp, asrc, adst, xw, ms, md, me,
      b1lo, b1hi, wll, wlh, whl, whh, as2lo, as2hi, ad2lo, ad2hi)


def _combine2_body(nm_ref, dn_ref, as_ref, ad_ref, xw_ref,
                   ms_ref, md_ref, b2_ref, h2_ref, lp_ref):
    den = dn_ref[...]
    S = jnp.maximum(ms_ref[0:1, :] + md_ref[0:1, :], 0.0)
    al = as_ref[...] + ad_ref[...]
    al = jnp.where(al >= 0.0, al, 0.2 * al)
    exl = jnp.exp(al - S)
    den_t = den + exl
    dinv = 1.0 / (den_t + 1e-16)
    rr = lax.broadcasted_iota(jnp.int32, (HW, C), 0)
    cc = lax.broadcasted_iota(jnp.int32, (HW, C), 1)
    Rm = jnp.where(rr % C == cc, 1.0 / H, 0.0).astype(_f32)
    h2 = b2_ref[...]
    for half, R in ((0, _bd_t(C, HW, 0)), (1, _bd_t(C, HW, HH))):
        exb = jnp.dot(exl, R, preferred_element_type=_f32)
        num_t = nm_ref[half] + xw_ref[half] * exb
        dinvb = jnp.dot(dinv, R, preferred_element_type=_f32)
        h2 = h2 + jnp.dot(num_t * dinvb, Rm, preferred_element_type=_f32)
    h2_ref[...] = h2
    m = jnp.max(h2, axis=1, keepdims=True)
    z = h2 - m
    lse = jnp.log(jnp.sum(jnp.exp(z), axis=1, keepdims=True))
    lp_ref[...] = z - lse


def _tc_combine2(nm, dn, asrc, adst, xw, ms, md, b2r):
    spec_cat = pl.BlockSpec((2, BN, HW), lambda i: (0, i, 0))
    spec_n16 = pl.BlockSpec((BN, C), lambda i: (i, 0))
    spec_m = pl.BlockSpec((8, C), lambda i: (0, 0))
    return pl.pallas_call(
        _combine2_body,
        grid=(GN,),
        in_specs=[
            spec_cat, spec_n16, spec_n16, spec_n16, spec_cat,
            spec_m, spec_m,
            pl.BlockSpec((1, C), lambda i: (0, 0)),
        ],
        out_specs=[spec_n16, spec_n16],
        out_shape=[
            jax.ShapeDtypeStruct((N, C), _f32),
            jax.ShapeDtypeStruct((N, C), _f32),
        ],
    )(nm, dn, asrc, adst, xw, ms, md, b2r)


def kernel(x, edge_index, edge_attr, W1, att_src1, att_dst1, W_edge1,
           att_edge1, bias1, W2, att_src2, att_dst2, bias2):
    src = edge_index[0]
    dst = edge_index[1]
    w1lo, w1hi = W1[:, :HW], W1[:, HW:]
    as1lo = att_src1[:HH].reshape(1, HW)
    as1hi = att_src1[HH:].reshape(1, HW)
    ad1lo = att_dst1[:HH].reshape(1, HW)
    ad1hi = att_dst1[HH:].reshape(1, HW)
    attef1 = att_edge1.reshape(1, HC)
    as2lo = att_src2[:HH].reshape(1, HW)
    as2hi = att_src2[HH:].reshape(1, HW)
    ad2lo = att_dst2[:HH].reshape(1, HW)
    ad2hi = att_dst2[HH:].reshape(1, HW)
    wll, wlh = W2[:HW, :HW], W2[:HW, HW:]
    whl, whh = W2[HW:, :HW], W2[HW:, HW:]
    b1lo = bias1[:HW].reshape(1, HW)
    b1hi = bias1[HW:].reshape(1, HW)

    xw1, asrc1, adst1, ms1, md1 = _tc_node_prologue(
        x, w1lo, w1hi, as1lo, as1hi, ad1lo, ad1hi)
    ae1, me1 = _tc_edge_prologue(edge_attr, W_edge1, attef1)
    num1, den1, loop1 = _make_sc_edge_pass(True)(src, dst, asrc1, adst1,
                                                 ae1, xw1, ms1, md1, me1)
    xw2, asrc2, adst2, ms2, md2 = _tc_combine1(
        num1, den1, loop1, asrc1, adst1, xw1, ms1, md1, me1,
        b1lo, b1hi, wll, wlh, whl, whh, as2lo, as2hi, ad2lo, ad2hi)
    num2, den2 = _make_sc_edge_pass(False)(src, dst, asrc2, adst2, xw2,
                                           ms2, md2)
    h2, lp = _tc_combine2(num2, den2, asrc2, adst2, xw2, ms2, md2,
                          bias2.reshape(1, C))
    return (h2, lp)


# speedup vs baseline: 39.4379x; 1.0007x over previous
import functools

import jax
import jax.numpy as jnp
from jax import lax
from jax.experimental import pallas as pl
from jax.experimental.pallas import tpu as pltpu
from jax.experimental.pallas import tpu_sc as plsc

N = 10000
E = 320000
NFEAT = 128
H = 12
C = 16
HC = H * C
HH = H // 2
HW = HH * C

BN = 400
GN = N // BN
BE = 8000
GE = E // BE

NSC = 2
NTILE = 16
NW = NSC * NTILE
CH = 80
EPT_A = E // NW
NCH_A = EPT_A // CH
EPT_B = E // NTILE
NCH_B = EPT_B // CH
RPT = N // NTILE
WB = 125
NWB = RPT // WB

_f32 = jnp.float32

_SC_PARAMS = pltpu.CompilerParams(
    use_tc_tiling_on_sc=False, needs_layout_passes=False)


def _bd(rows, cols, shift):
    r = lax.broadcasted_iota(jnp.int32, (rows, cols), 0)
    c = lax.broadcasted_iota(jnp.int32, (rows, cols), 1)
    return (r // C + shift == c).astype(_f32)


def _bd_t(rows, cols, shift):
    r = lax.broadcasted_iota(jnp.int32, (rows, cols), 0)
    c = lax.broadcasted_iota(jnp.int32, (rows, cols), 1)
    return (c // C + shift == r).astype(_f32)


def _node_prologue_body(x_ref, wlo_ref, whi_ref, aslo_ref, ashi_ref,
                        adlo_ref, adhi_ref,
                        xw_ref, as_ref, ad_ref, ms_ref, md_ref):
    i = pl.program_id(0)
    xb = x_ref[...]
    xwlo = jnp.dot(xb, wlo_ref[...], preferred_element_type=_f32)
    xwhi = jnp.dot(xb, whi_ref[...], preferred_element_type=_f32)
    xw_ref[0] = xwlo
    xw_ref[1] = xwhi
    Blo = _bd(HW, C, 0)
    Bhi = _bd(HW, C, HH)
    a_s = (jnp.dot(xwlo * aslo_ref[...], Blo, preferred_element_type=_f32)
           + jnp.dot(xwhi * ashi_ref[...], Bhi, preferred_element_type=_f32))
    a_d = (jnp.dot(xwlo * adlo_ref[...], Blo, preferred_element_type=_f32)
           + jnp.dot(xwhi * adhi_ref[...], Bhi, preferred_element_type=_f32))
    as_ref[...] = a_s
    ad_ref[...] = a_d
    bs = jnp.broadcast_to(jnp.max(a_s, axis=0, keepdims=True), (8, C))
    bd = jnp.broadcast_to(jnp.max(a_d, axis=0, keepdims=True), (8, C))

    @pl.when(i == 0)
    def _():
        ms_ref[...] = bs
        md_ref[...] = bd

    @pl.when(i > 0)
    def _():
        ms_ref[...] = jnp.maximum(ms_ref[...], bs)
        md_ref[...] = jnp.maximum(md_ref[...], bd)


def _tc_node_prologue(x, wlo, whi, aslo, ashi, adlo, adhi):
    w_spec = pl.BlockSpec((NFEAT, HW), lambda i: (0, 0))
    v_spec = pl.BlockSpec((1, HW), lambda i: (0, 0))
    return pl.pallas_call(
        _node_prologue_body,
        grid=(GN,),
        in_specs=[
            pl.BlockSpec((BN, NFEAT), lambda i: (i, 0)),
            w_spec, w_spec, v_spec, v_spec, v_spec, v_spec,
        ],
        out_specs=[
            pl.BlockSpec((2, BN, HW), lambda i: (0, i, 0)),
            pl.BlockSpec((BN, C), lambda i: (i, 0)),
            pl.BlockSpec((BN, C), lambda i: (i, 0)),
            pl.BlockSpec((8, C), lambda i: (0, 0)),
            pl.BlockSpec((8, C), lambda i: (0, 0)),
        ],
        out_shape=[
            jax.ShapeDtypeStruct((2, N, HW), _f32),
            jax.ShapeDtypeStruct((N, C), _f32),
            jax.ShapeDtypeStruct((N, C), _f32),
            jax.ShapeDtypeStruct((8, C), _f32),
            jax.ShapeDtypeStruct((8, C), _f32),
        ],
    )(x, wlo, whi, aslo, ashi, adlo, adhi)


def _edge_prologue_body(ea_ref, we_ref, atte_ref, ae_ref, me_ref):
    i = pl.program_id(0)
    B = _bd(HC, C, 0)
    Me = jnp.dot(we_ref[...] * atte_ref[...], B, preferred_element_type=_f32)
    ae = jnp.dot(ea_ref[...], Me, preferred_element_type=_f32)
    col = lax.broadcasted_iota(jnp.int32, (BE, C), 1)
    ae = ae + jnp.where(col == H, 1.0, 0.0).astype(_f32)
    ae_ref[...] = ae
    bm = jnp.broadcast_to(jnp.max(ae, axis=0, keepdims=True), (8, C))

    @pl.when(i == 0)
    def _():
        me_ref[...] = bm

    @pl.when(i > 0)
    def _():
        me_ref[...] = jnp.maximum(me_ref[...], bm)


def _tc_edge_prologue(ea, we, attef):
    return pl.pallas_call(
        _edge_prologue_body,
        grid=(GE,),
        in_specs=[
            pl.BlockSpec((BE, C), lambda i: (i, 0)),
            pl.BlockSpec((C, HC), lambda i: (0, 0)),
            pl.BlockSpec((1, HC), lambda i: (0, 0)),
        ],
        out_specs=[
            pl.BlockSpec((BE, C), lambda i: (i, 0)),
            pl.BlockSpec((8, C), lambda i: (0, 0)),
        ],
        out_shape=[
            jax.ShapeDtypeStruct((E, C), _f32),
            jax.ShapeDtypeStruct((8, C), _f32),
        ],
    )(ea, we, attef)


def _sc_edge_pass_body(has_edge, *refs):
    if has_edge:
        (ei_hbm, as_hbm, ad_hbm, ae_hbm, xw_hbm, ms_hbm, md_hbm,
         me_hbm, num_out, den_out, loop_out,
         idx0, idx1, idxc0, idxc1,
         rs0, rs1, rd0, rd1, re0, re1, rec0, rec1,
         xwr0, xwr1, outr0, outr1, exr0, exr1,
         m1, m2, m3, accn, accd, accl,
         gsem0, gsem1, ssem0, ssem1, isem0, isem1) = refs
        re_ = (re0, re1)
        rec_ = (rec0, rec1)
    else:
        (ei_hbm, as_hbm, ad_hbm, xw_hbm, ms_hbm, md_hbm,
         num_out, den_out,
         idx0, idx1, idxc0, idxc1,
         rs0, rs1, rd0, rd1, xwr0, xwr1, outr0, outr1, exr0, exr1,
         m1, m2, m3, accn, accd,
         gsem0, gsem1, ssem0, ssem1, isem0, isem1) = refs
        re_ = (None, None)
        rec_ = (None, None)
        accl = None
        loop_out = None
    idx_ = (idx0, idx1)
    idxc_ = (idxc0, idxc1)
    rs_ = (rs0, rs1)
    rd_ = (rd0, rd1)
    xwr_ = (xwr0, xwr1)
    outr_ = (outr0, outr1)
    exr_ = (exr0, exr1)
    gsem_ = (gsem0, gsem1)
    ssem_ = (ssem0, ssem1)
    isem_ = (isem0, isem1)
    c = lax.axis_index("c")
    s = lax.axis_index("s")

    pltpu.sync_copy(ms_hbm, m1)
    pltpu.sync_copy(md_hbm, m2)
    S = m1[0, :] + m2[0, :]
    if has_edge:
        pltpu.sync_copy(me_hbm, m3)
        S = S + m3[0, :]
    S = jnp.maximum(S, 0.0)
    idxh_ = [jnp.full((C,), h, jnp.int32) + c * HH for h in range(HH)]

    def zrow(i, _):
        for jj in range(HH):
            xwr0[i, pl.ds(jj * C, C)] = jnp.zeros((C,), _f32)
        rs0[i, :] = jnp.zeros((C,), _f32)
        return 0

    lax.fori_loop(0, CH, zrow, 0)
    segs = [(k * CH, CH) for k in range(RPT // CH)] + [
        (RPT - RPT % CH, RPT % CH)]
    for off, sz in segs:
        sl = pl.ds(s * RPT + off, sz)
        pltpu.sync_copy(xwr0.at[pl.ds(0, sz)], accn.at[sl])
        pltpu.sync_copy(rs0.at[pl.ds(0, sz)], accd.at[sl])
        if has_edge:
            pltpu.sync_copy(rs0.at[pl.ds(0, sz)], accl.at[sl])
    plsc.subcore_barrier()

    def start_idx(j, b):
        base = s * EPT_B + j * CH
        pltpu.async_copy(ei_hbm.at[:, pl.ds(base, CH)], idx_[b], isem_[b])

    def wait_idx(b):
        pltpu.make_async_copy(
            ei_hbm.at[:, pl.ds(0, CH)], idx_[b], isem_[b]).wait()

    def start_gathers(j, b):
        base = s * EPT_B + j * CH
        pltpu.async_copy(as_hbm.at[idx_[b].at[0]], rs_[b], gsem_[b])
        pltpu.async_copy(ad_hbm.at[idx_[b].at[1]], rd_[b], gsem_[b])
        pltpu.async_copy(xw_hbm.at[c].at[idx_[b].at[0]], xwr_[b], gsem_[b])
        if has_edge:
            pltpu.async_copy(ae_hbm.at[pl.ds(base, CH)], re_[b], gsem_[b])

    def wait_gathers(b):
        pltpu.make_async_copy(
            as_hbm.at[idx_[b].at[0]], rs_[b], gsem_[b]).wait()
        pltpu.make_async_copy(
            ad_hbm.at[idx_[b].at[1]], rd_[b], gsem_[b]).wait()
        pltpu.make_async_copy(
            xw_hbm.at[c].at[idx_[b].at[0]], xwr_[b], gsem_[b]).wait()
        if has_edge:
            pltpu.make_async_copy(
                ae_hbm.at[pl.ds(0, CH)], re_[b], gsem_[b]).wait()

    def wait_scatters(b):
        pltpu.make_async_copy(
            outr_[b], accn.at[idxc_[b]], ssem_[b]).wait()

        @pl.when(c == 0)
        def _():
            pltpu.make_async_copy(
                exr_[b], accd.at[idxc_[b]], ssem_[b]).wait()

        if has_edge:
            @pl.when(c == 1)
            def _():
                pltpu.make_async_copy(
                    rec_[b], accl.at[idxc_[b]], ssem_[b]).wait()

    start_idx(0, 0)
    start_idx(1, 1)
    wait_idx(0)
    start_gathers(0, 0)

    def outer(jo, _):
        for b in (0, 1):
            j = 2 * jo + b
            @pl.when(j + 1 < NCH_B)
            def _():
                wait_idx(1 - b)
                start_gathers(j + 1, 1 - b)

            wait_gathers(b)

            @pl.when(jo > 0)
            def _():
                wait_scatters(b)

            for k in range(CH // C):
                idxc_[b][pl.ds(k * C, C)] = idx_[b][1, pl.ds(k * C, C)]

            @pl.when(j + 2 < NCH_B)
            def _():
                start_idx(j + 2, b)

            rs, rd, re = rs_[b], rd_[b], re_[b]
            xwr, outr, exr = xwr_[b], outr_[b], exr_[b]

            def edge_ex(e, _):
                a = rs[e, :] + rd[e, :]
                if has_edge:
                    a = a + re[e, :]
                a = jnp.where(a >= 0.0, a, 0.2 * a)
                exr[e, :] = jnp.exp(a - S)
                return 0

            lax.fori_loop(0, CH, edge_ex, 0, unroll=4)

            def edge_mul(e, _):
                idxe = jnp.full((C,), e, jnp.int32)
                for h in range(HH):
                    bc = plsc.load_gather(exr, [idxe, idxh_[h]])
                    outr[e, pl.ds(h * C, C)] = xwr[e, pl.ds(h * C, C)] * bc
                return 0

            lax.fori_loop(0, CH, edge_mul, 0, unroll=8)
            pltpu.async_copy(outr, accn.at[idxc_[b]], ssem_[b], add=True)

            @pl.when(c == 0)
            def _():
                pltpu.async_copy(exr, accd.at[idxc_[b]], ssem_[b], add=True)

            if has_edge:
                @pl.when(c == 1)
                def _():
                    def cprow(e, _):
                        rec_[b][e, :] = re[e, :]
                        return 0

                    lax.fori_loop(0, CH, cprow, 0, unroll=8)
                    pltpu.async_copy(
                        rec_[b], accl.at[idxc_[b]], ssem_[b], add=True)

        return 0

    lax.fori_loop(0, NCH_B // 2, outer, 0)
    wait_scatters(0)
    wait_scatters(1)
    plsc.subcore_barrier()
    for off, sz in segs:
        sl = pl.ds(s * RPT + off, sz)
        pltpu.sync_copy(accn.at[sl], xwr0.at[pl.ds(0, sz)])
        pltpu.sync_copy(xwr0.at[pl.ds(0, sz)], num_out.at[c, sl])

    @pl.when(c == 0)
    def _():
        for off, sz in segs:
            sl = pl.ds(s * RPT + off, sz)
            pltpu.sync_copy(accd.at[sl], rs0.at[pl.ds(0, sz)])
            pltpu.sync_copy(rs0.at[pl.ds(0, sz)], den_out.at[sl])

    if has_edge:
        @pl.when(c == 1)
        def _():
            for off, sz in segs:
                sl = pl.ds(s * RPT + off, sz)
                pltpu.sync_copy(accl.at[sl], rd0.at[pl.ds(0, sz)])
                pltpu.sync_copy(rd0.at[pl.ds(0, sz)], loop_out.at[sl])


@functools.cache
def _make_sc_edge_pass(has_edge):
    scratch = [
        pltpu.VMEM((2, CH), jnp.int32),
        pltpu.VMEM((2, CH), jnp.int32),
        pltpu.VMEM((CH,), jnp.int32),
        pltpu.VMEM((CH,), jnp.int32),
        pltpu.VMEM((CH, C), _f32),
        pltpu.VMEM((CH, C), _f32),
        pltpu.VMEM((CH, C), _f32),
        pltpu.VMEM((CH, C), _f32),
    ]
    if has_edge:
        scratch += [pltpu.VMEM((CH, C), _f32),
                    pltpu.VMEM((CH, C), _f32),
                    pltpu.VMEM((CH, C), _f32),
                    pltpu.VMEM((CH, C), _f32)]
    scratch += [
        pltpu.VMEM((CH, HW), _f32),
        pltpu.VMEM((CH, HW), _f32),
        pltpu.VMEM((CH, HW), _f32),
        pltpu.VMEM((CH, HW), _f32),
        pltpu.VMEM((CH, C), _f32),
        pltpu.VMEM((CH, C), _f32),
        pltpu.VMEM((8, C), _f32),
        pltpu.VMEM((8, C), _f32),
        pltpu.VMEM((8, C), _f32),
        pltpu.VMEM_SHARED((N, HW), _f32),
        pltpu.VMEM_SHARED((N, C), _f32),
    ]
    if has_edge:
        scratch.append(pltpu.VMEM_SHARED((N, C), _f32))
    scratch += [
        pltpu.SemaphoreType.DMA,
        pltpu.SemaphoreType.DMA,
        pltpu.SemaphoreType.DMA,
        pltpu.SemaphoreType.DMA,
        pltpu.SemaphoreType.DMA,
        pltpu.SemaphoreType.DMA,
    ]
    outs = [
        jax.ShapeDtypeStruct((NSC, N, HW), _f32),
        jax.ShapeDtypeStruct((N, C), _f32),
    ]
    if has_edge:
        outs.append(jax.ShapeDtypeStruct((N, C), _f32))
    return pl.kernel(
        functools.partial(_sc_edge_pass_body, has_edge),
        out_type=tuple(outs),
        mesh=plsc.VectorSubcoreMesh(core_axis_name="c", subcore_axis_name="s",
                                    num_cores=NSC, num_subcores=NTILE),
        scratch_types=scratch,
        compiler_params=_SC_PARAMS,
    )


def _combine1_body(nm_ref, dn_ref, lp_ref,
                   as_ref, ad_ref, xw_ref, ms_ref, md_ref, me_ref,
                   b1lo_ref, b1hi_ref, wll_ref, wlh_ref, whl_ref, whh_ref,
                   as2lo_ref, as2hi_ref, ad2lo_ref, ad2hi_ref,
                   xw2_ref, as2_ref, ad2_ref, ms2_ref, md2_ref):
    i = pl.program_id(0)
    den = dn_ref[...]
    ls = lp_ref[...]
    onehot12 = jnp.where(
        lax.broadcasted_iota(jnp.int32, (1, C), 1) == H, 1.0, 0.0).astype(_f32)
    cnt = jnp.sum(ls * onehot12, axis=1, keepdims=True)
    lae = ls / jnp.maximum(cnt, 1.0)
    S = jnp.maximum(ms_ref[0:1, :] + md_ref[0:1, :] + me_ref[0:1, :], 0.0)
    al = as_ref[...] + ad_ref[...] + lae
    al = jnp.where(al >= 0.0, al, 0.2 * al)
    exl = jnp.exp(al - S)
    Rlo = _bd_t(C, HW, 0)
    Rhi = _bd_t(C, HW, HH)
    den_t = den + exl
    dinv = 1.0 / (den_t + 1e-16)
    hs = []
    for half, R, b1 in ((0, Rlo, b1lo_ref), (1, Rhi, b1hi_ref)):
        exb = jnp.dot(exl, R, preferred_element_type=_f32)
        num_t = nm_ref[half] + xw_ref[half] * exb
        dinvb = jnp.dot(dinv, R, preferred_element_type=_f32)
        hv = num_t * dinvb + b1[...]
        hs.append(jnp.where(hv > 0.0, hv, jnp.exp(hv) - 1.0))
    xw2lo = (jnp.dot(hs[0], wll_ref[...], preferred_element_type=_f32)
             + jnp.dot(hs[1], whl_ref[...], preferred_element_type=_f32))
    xw2hi = (jnp.dot(hs[0], wlh_ref[...], preferred_element_type=_f32)
             + jnp.dot(hs[1], whh_ref[...], preferred_element_type=_f32))
    xw2_ref[0] = xw2lo
    xw2_ref[1] = xw2hi
    Blo = _bd(HW, C, 0)
    Bhi = _bd(HW, C, HH)
    a_s2 = (jnp.dot(xw2lo * as2lo_ref[...], Blo, preferred_element_type=_f32)
            + jnp.dot(xw2hi * as2hi_ref[...], Bhi,
                      preferred_element_type=_f32))
    a_d2 = (jnp.dot(xw2lo * ad2lo_ref[...], Blo, preferred_element_type=_f32)
            + jnp.dot(xw2hi * ad2hi_ref[...], Bhi,
                      preferred_element_type=_f32))
    as2_ref[...] = a_s2
    ad2_ref[...] = a_d2
    bs = jnp.broadcast_to(jnp.max(a_s2, axis=0, keepdims=True), (8, C))
    bd = jnp.broadcast_to(jnp.max(a_d2, axis=0, keepdims=True), (8, C))

    @pl.when(i == 0)
    def _():
        ms2_ref[...] = bs
        md2_ref[...] = bd

    @pl.when(i > 0)
    def _():
        ms2_ref[...] = jnp.maximum(ms2_ref[...], bs)
        md2_ref[...] = jnp.maximum(md2_ref[...], bd)


def _tc_combine1(nm, dn, lp, asrc, adst, xw, ms, md, me,
                 b1lo, b1hi, wll, wlh, whl, whh,
                 as2lo, as2hi, ad2lo, ad2hi):
    spec_cat = pl.BlockSpec((2, BN, HW), lambda i: (0, i, 0))
    spec_n16 = pl.BlockSpec((BN, C), lambda i: (i, 0))
    spec_m = pl.BlockSpec((8, C), lambda i: (0, 0))
    spec_v = pl.BlockSpec((1, HW), lambda i: (0, 0))
    spec_w = pl.BlockSpec((HW, HW), lambda i: (0, 0))
    return pl.pallas_call(
        _combine1_body,
        grid=(GN,),
        in_specs=[
            spec_cat, spec_n16, spec_n16,
            spec_n16, spec_n16, spec_cat,
            spec_m, spec_m, spec_m,
            spec_v, spec_v, spec_w, spec_w, spec_w, spec_w,
            spec_v, spec_v, spec_v, spec_v,
        ],
        out_specs=[spec_cat, spec_n16, spec_n16, spec_m, spec_m],
        out_shape=[
            jax.ShapeDtypeStruct((2, N, HW), _f32),
            jax.ShapeDtypeStruct((N, C), _f32),
            jax.ShapeDtypeStruct((N, C), _f32),
            jax.ShapeDtypeStruct((8, C), _f32),
            jax.ShapeDtypeStruct((8, C), _f32),
        ],
    )(nm, dn, lp, asrc, adst, xw, ms, md, me,
      b1lo, b1hi, wll, wlh, whl, whh, as2lo, as2hi, ad2lo, ad2hi)


def _combine2_body(nm_ref, dn_ref, as_ref, ad_ref, xw_ref,
                   ms_ref, md_ref, b2_ref, h2_ref, lp_ref):
    den = dn_ref[...]
    S = jnp.maximum(ms_ref[0:1, :] + md_ref[0:1, :], 0.0)
    al = as_ref[...] + ad_ref[...]
    al = jnp.where(al >= 0.0, al, 0.2 * al)
    exl = jnp.exp(al - S)
    den_t = den + exl
    dinv = 1.0 / (den_t + 1e-16)
    rr = lax.broadcasted_iota(jnp.int32, (HW, C), 0)
    cc = lax.broadcasted_iota(jnp.int32, (HW, C), 1)
    Rm = jnp.where(rr % C == cc, 1.0 / H, 0.0).astype(_f32)
    h2 = b2_ref[...]
    for half, R in ((0, _bd_t(C, HW, 0)), (1, _bd_t(C, HW, HH))):
        exb = jnp.dot(exl, R, preferred_element_type=_f32)
        num_t = nm_ref[half] + xw_ref[half] * exb
        dinvb = jnp.dot(dinv, R, preferred_element_type=_f32)
        h2 = h2 + jnp.dot(num_t * dinvb, Rm, preferred_element_type=_f32)
    h2_ref[...] = h2
    m = jnp.max(h2, axis=1, keepdims=True)
    z = h2 - m
    lse = jnp.log(jnp.sum(jnp.exp(z), axis=1, keepdims=True))
    lp_ref[...] = z - lse


def _tc_combine2(nm, dn, asrc, adst, xw, ms, md, b2r):
    spec_cat = pl.BlockSpec((2, BN, HW), lambda i: (0, i, 0))
    spec_n16 = pl.BlockSpec((BN, C), lambda i: (i, 0))
    spec_m = pl.BlockSpec((8, C), lambda i: (0, 0))
    return pl.pallas_call(
        _combine2_body,
        grid=(GN,),
        in_specs=[
            spec_cat, spec_n16, spec_n16, spec_n16, spec_cat,
            spec_m, spec_m,
            pl.BlockSpec((1, C), lambda i: (0, 0)),
        ],
        out_specs=[spec_n16, spec_n16],
        out_shape=[
            jax.ShapeDtypeStruct((N, C), _f32),
            jax.ShapeDtypeStruct((N, C), _f32),
        ],
    )(nm, dn, asrc, adst, xw, ms, md, b2r)


def kernel(x, edge_index, edge_attr, W1, att_src1, att_dst1, W_edge1,
           att_edge1, bias1, W2, att_src2, att_dst2, bias2):
    w1lo, w1hi = W1[:, :HW], W1[:, HW:]
    as1lo = att_src1[:HH].reshape(1, HW)
    as1hi = att_src1[HH:].reshape(1, HW)
    ad1lo = att_dst1[:HH].reshape(1, HW)
    ad1hi = att_dst1[HH:].reshape(1, HW)
    attef1 = att_edge1.reshape(1, HC)
    as2lo = att_src2[:HH].reshape(1, HW)
    as2hi = att_src2[HH:].reshape(1, HW)
    ad2lo = att_dst2[:HH].reshape(1, HW)
    ad2hi = att_dst2[HH:].reshape(1, HW)
    wll, wlh = W2[:HW, :HW], W2[:HW, HW:]
    whl, whh = W2[HW:, :HW], W2[HW:, HW:]
    b1lo = bias1[:HW].reshape(1, HW)
    b1hi = bias1[HW:].reshape(1, HW)

    xw1, asrc1, adst1, ms1, md1 = _tc_node_prologue(
        x, w1lo, w1hi, as1lo, as1hi, ad1lo, ad1hi)
    ae1, me1 = _tc_edge_prologue(edge_attr, W_edge1, attef1)
    num1, den1, loop1 = _make_sc_edge_pass(True)(edge_index, asrc1, adst1,
                                                 ae1, xw1, ms1, md1, me1)
    xw2, asrc2, adst2, ms2, md2 = _tc_combine1(
        num1, den1, loop1, asrc1, adst1, xw1, ms1, md1, me1,
        b1lo, b1hi, wll, wlh, whl, whh, as2lo, as2hi, ad2lo, ad2hi)
    num2, den2 = _make_sc_edge_pass(False)(edge_index, asrc2, adst2, xw2,
                                           ms2, md2)
    h2, lp = _tc_combine2(num2, den2, asrc2, adst2, xw2, ms2, md2,
                          bias2.reshape(1, C))
    return (h2, lp)
